# Initial kernel scaffold; baseline (speedup 1.0000x reference)
import jax
import jax.numpy as jnp
from jax.experimental import pallas as pl


def kernel(x, pos, edge_index, W_lin, W_src, W_dst, pW1, pb1, pg1, pbe1, pW2, pb2, pg2, pbe2, aW1, ab1, ag1, abe1, aW2, ab2, ag2, abe2, up_W, up_b):
    raise NotImplementedError("write your pallas kernel here")


# speedup vs baseline: 1.7784x; 1.7784x over previous
import functools

import jax
import jax.numpy as jnp
from jax import lax
from jax.experimental import pallas as pl
from jax.experimental.pallas import tpu as pltpu
from jax.experimental.pallas import tpu_sc as plsc

_N = 10000
_E = 320000
_C = 128

_EP = 327680
_B = 4096
_GRID = _EP // _B

_GCH = 128
_GNC = _EP // _GCH
_CPW = _GNC // 32

_SPS = _GNC // 16
_ZR = 624

_f32 = jnp.float32


def _prep_body(x_ref, pos_ref, w_ref, w1_ref, o_ref, p_ref):
    o_ref[...] = jnp.dot(x_ref[...], w_ref[...],
                         preferred_element_type=_f32)
    p_ref[...] = jnp.dot(pos_ref[...], w1_ref[...],
                         preferred_element_type=_f32)


def _prep(x, pos16, wcat, w1p):
    return pl.pallas_call(
        _prep_body,
        grid=(5,),
        in_specs=[pl.BlockSpec((2000, _C), lambda i: (i, 0)),
                  pl.BlockSpec((2000, 16), lambda i: (i, 0)),
                  pl.BlockSpec((_C, 3 * _C), lambda i: (0, 0)),
                  pl.BlockSpec((16, _C), lambda i: (0, 0))],
        out_specs=[pl.BlockSpec((2000, 3 * _C), lambda i: (i, 0)),
                   pl.BlockSpec((2000, _C), lambda i: (i, 0))],
        out_shape=[jax.ShapeDtypeStruct((_N, 3 * _C), _f32),
                   jax.ShapeDtypeStruct((_N, _C), _f32)],
    )(x, pos16, wcat, w1p)


def _edge_mask():
    rows = lax.broadcasted_iota(jnp.int32, (_B, 1), 0) + pl.program_id(0) * _B
    return (rows < _E).astype(_f32)


def _acc_stats(st_ref, h):
    m = _edge_mask()
    hm = h * m
    blk = jnp.concatenate([jnp.sum(hm, 0, keepdims=True),
                           jnp.sum(hm * h, 0, keepdims=True)])

    @pl.when(pl.program_id(0) == 0)
    def _():
        st_ref[...] = jnp.zeros_like(st_ref)

    st_ref[...] += blk


def _p1_body(gpd_ref, gps_ref, b_ref, st_ref):
    h = gpd_ref[...] - gps_ref[...] + b_ref[...]
    _acc_stats(st_ref, h)


def _pass1(gpd, gps, b1):
    return pl.pallas_call(
        _p1_body,
        grid=(_GRID,),
        in_specs=[pl.BlockSpec((_B, _C), lambda i: (i, 0)),
                  pl.BlockSpec((_B, _C), lambda i: (i, 0)),
                  pl.BlockSpec((1, _C), lambda i: (0, 0))],
        out_specs=pl.BlockSpec((2, _C), lambda i: (0, 0)),
        out_shape=jax.ShapeDtypeStruct((2, _C), _f32),
    )(gpd, gps, b1)


def _p2_body(gpd_ref, gps_ref, s1_ref, w2_ref, b2_ref,
             h2_ref, st_ref):
    r = jnp.maximum((gpd_ref[...] - gps_ref[...]) * s1_ref[0:1]
                    + s1_ref[1:2], 0.0)
    h2 = jnp.dot(r, w2_ref[...], preferred_element_type=_f32) + b2_ref[...]
    h2_ref[...] = h2
    _acc_stats(st_ref, h2)


def _pass2(gpd, gps, s1, w2, b2):
    return pl.pallas_call(
        _p2_body,
        grid=(_GRID,),
        in_specs=[pl.BlockSpec((_B, _C), lambda i: (i, 0)),
                  pl.BlockSpec((_B, _C), lambda i: (i, 0)),
                  pl.BlockSpec((2, _C), lambda i: (0, 0)),
                  pl.BlockSpec((_C, _C), lambda i: (0, 0)),
                  pl.BlockSpec((1, _C), lambda i: (0, 0))],
        out_specs=[pl.BlockSpec((_B, _C), lambda i: (i, 0)),
                   pl.BlockSpec((2, _C), lambda i: (0, 0))],
        out_shape=[jax.ShapeDtypeStruct((_EP, _C), _f32),
                   jax.ShapeDtypeStruct((2, _C), _f32)],
    )(gpd, gps, s1, w2, b2)


def _p3_body(h2_ref, gad_ref, gas_ref, s2_ref, w_ref, b_ref,
             h3_ref, st_ref):
    delta = jnp.maximum(h2_ref[...] * s2_ref[0:1] + s2_ref[1:2], 0.0)
    a0 = gad_ref[...] - gas_ref[...] + delta
    h3 = jnp.dot(a0, w_ref[...], preferred_element_type=_f32) + b_ref[...]
    h3_ref[...] = h3
    _acc_stats(st_ref, h3)


def _pass3(h2, gad, gas, s2, w, b):
    return pl.pallas_call(
        _p3_body,
        grid=(_GRID,),
        in_specs=[pl.BlockSpec((_B, _C), lambda i: (i, 0)),
                  pl.BlockSpec((_B, _C), lambda i: (i, 0)),
                  pl.BlockSpec((_B, _C), lambda i: (i, 0)),
                  pl.BlockSpec((2, _C), lambda i: (0, 0)),
                  pl.BlockSpec((_C, _C), lambda i: (0, 0)),
                  pl.BlockSpec((1, _C), lambda i: (0, 0))],
        out_specs=[pl.BlockSpec((_B, _C), lambda i: (i, 0)),
                   pl.BlockSpec((2, _C), lambda i: (0, 0))],
        out_shape=[jax.ShapeDtypeStruct((_EP, _C), _f32),
                   jax.ShapeDtypeStruct((2, _C), _f32)],
    )(h2, gad, gas, s2, w, b)


def _p4_body(h3_ref, s3_ref, w_ref, b_ref, h4_ref, st_ref):
    s = jnp.maximum(h3_ref[...] * s3_ref[0:1] + s3_ref[1:2], 0.0)
    h4 = jnp.dot(s, w_ref[...], preferred_element_type=_f32) + b_ref[...]
    h4_ref[...] = h4
    _acc_stats(st_ref, h4)


def _pass4(h3, s3, w, b):
    return pl.pallas_call(
        _p4_body,
        grid=(_GRID,),
        in_specs=[pl.BlockSpec((_B, _C), lambda i: (i, 0)),
                  pl.BlockSpec((2, _C), lambda i: (0, 0)),
                  pl.BlockSpec((_C, _C), lambda i: (0, 0)),
                  pl.BlockSpec((1, _C), lambda i: (0, 0))],
        out_specs=[pl.BlockSpec((_B, _C), lambda i: (i, 0)),
                   pl.BlockSpec((2, _C), lambda i: (0, 0))],
        out_shape=[jax.ShapeDtypeStruct((_EP, _C), _f32),
                   jax.ShapeDtypeStruct((2, _C), _f32)],
    )(h3, s3, w, b)


def _p5_body(h4_ref, h2_ref, gv_ref, s4_ref, s2_ref, e_ref, w_ref):
    m = _edge_mask()
    alpha = jnp.maximum(h4_ref[...] * s4_ref[0:1] + s4_ref[1:2], 0.0)
    e = jnp.exp(alpha) * m
    delta = jnp.maximum(h2_ref[...] * s2_ref[0:1] + s2_ref[1:2], 0.0)
    e_ref[...] = e
    w_ref[...] = e * (gv_ref[...] + delta)


def _pass5(h4, h2, gv, s4, s2):
    return pl.pallas_call(
        _p5_body,
        grid=(_GRID,),
        in_specs=[pl.BlockSpec((_B, _C), lambda i: (i, 0)),
                  pl.BlockSpec((_B, _C), lambda i: (i, 0)),
                  pl.BlockSpec((_B, _C), lambda i: (i, 0)),
                  pl.BlockSpec((2, _C), lambda i: (0, 0)),
                  pl.BlockSpec((2, _C), lambda i: (0, 0))],
        out_specs=[pl.BlockSpec((_B, _C), lambda i: (i, 0)),
                   pl.BlockSpec((_B, _C), lambda i: (i, 0))],
        out_shape=[jax.ShapeDtypeStruct((_EP, _C), _f32),
                   jax.ShapeDtypeStruct((_EP, _C), _f32)],
    )(h4, h2, gv, s4, s2)


def _fin_body(s_ref, acc_ref, x_ref, w_ref, b_ref, o_ref):
    o = acc_ref[...] / (s_ref[...] + 1e-16)
    o_ref[...] = jnp.dot(o, w_ref[...],
                         preferred_element_type=_f32) + b_ref[...] + x_ref[...]


def _final(s, acc, x, w, b):
    return pl.pallas_call(
        _fin_body,
        grid=(5,),
        in_specs=[pl.BlockSpec((2000, _C), lambda i: (i, 0)),
                  pl.BlockSpec((2000, _C), lambda i: (i, 0)),
                  pl.BlockSpec((2000, _C), lambda i: (i, 0)),
                  pl.BlockSpec((_C, _C), lambda i: (0, 0)),
                  pl.BlockSpec((1, _C), lambda i: (0, 0))],
        out_specs=pl.BlockSpec((2000, _C), lambda i: (i, 0)),
        out_shape=jax.ShapeDtypeStruct((_N, _C), _f32),
    )(s, acc, x, w, b)


def _sc_gather(a_dst_t, a_src_t, v_t, p_t, idxd2, idxs2):
    mesh = plsc.VectorSubcoreMesh(core_axis_name="c", subcore_axis_name="s")
    out_type = [jax.ShapeDtypeStruct((_EP, _C), _f32) for _ in range(5)]

    @functools.partial(
        pl.kernel, mesh=mesh, out_type=out_type,
        scratch_types=[pltpu.VMEM((_CPW, _GCH), jnp.int32),
                       pltpu.VMEM((_CPW, _GCH), jnp.int32),
                       pltpu.VMEM((_GCH, _C), _f32),
                       pltpu.SemaphoreType.DMA])
    def k(adst_h, asrc_h, v_h, p_h, idxd_h, idxs_h,
          gad_h, gas_h, gv_h, gpd_h, gps_h,
          idxd_v, idxs_v, rbuf, sem):
        w = lax.axis_index("s") * 2 + lax.axis_index("c")
        base = w * _CPW
        pltpu.sync_copy(idxd_h.at[pl.ds(base, _CPW)], idxd_v)
        pltpu.sync_copy(idxs_h.at[pl.ds(base, _CPW)], idxs_v)

        @pl.loop(0, _CPW)
        def _(kk):
            row0 = (base + kk) * _GCH
            pltpu.async_copy(adst_h.at[idxd_v.at[kk]], rbuf, sem).wait()
            pltpu.sync_copy(rbuf, gad_h.at[pl.ds(row0, _GCH)])
            pltpu.async_copy(asrc_h.at[idxs_v.at[kk]], rbuf, sem).wait()
            pltpu.sync_copy(rbuf, gas_h.at[pl.ds(row0, _GCH)])
            pltpu.async_copy(v_h.at[idxs_v.at[kk]], rbuf, sem).wait()
            pltpu.sync_copy(rbuf, gv_h.at[pl.ds(row0, _GCH)])
            pltpu.async_copy(p_h.at[idxd_v.at[kk]], rbuf, sem).wait()
            pltpu.sync_copy(rbuf, gpd_h.at[pl.ds(row0, _GCH)])
            pltpu.async_copy(p_h.at[idxs_v.at[kk]], rbuf, sem).wait()
            pltpu.sync_copy(rbuf, gps_h.at[pl.ds(row0, _GCH)])

    return k(a_dst_t, a_src_t, v_t, p_t, idxd2, idxs2)


def _sc_scatter(e_arr, w_arr, idx2, zeros):
    mesh = plsc.VectorSubcoreMesh(core_axis_name="c", subcore_axis_name="s")
    out_type = [jax.ShapeDtypeStruct((_N, _C), _f32),
                jax.ShapeDtypeStruct((_N, _C), _f32)]

    @functools.partial(
        pl.kernel, mesh=mesh, out_type=out_type,
        scratch_types=[pltpu.VMEM_SHARED((_N, _C), _f32),
                       pltpu.VMEM((_SPS, _GCH), jnp.int32),
                       pltpu.VMEM((_GCH, _C), _f32)])
    def k(e_h, w_h, idx_h, z_h, s_out, a_out, spm, idx_v, rbuf):
        c = lax.axis_index("c")
        sid = lax.axis_index("s")
        pltpu.sync_copy(z_h.at[pl.ds(sid * _ZR, _ZR)],
                        spm.at[pl.ds(sid * _ZR, _ZR)])

        @pl.when(sid == 15)
        def _():
            pltpu.sync_copy(z_h.at[pl.ds(16 * _ZR, _N - 16 * _ZR)],
                            spm.at[pl.ds(16 * _ZR, _N - 16 * _ZR)])

        pltpu.sync_copy(idx_h.at[pl.ds(sid * _SPS, _SPS)], idx_v)
        plsc.subcore_barrier()

        def scat(src_h):
            @pl.loop(0, _SPS)
            def _(kk):
                row0 = (sid * _SPS + kk) * _GCH
                pltpu.sync_copy(src_h.at[pl.ds(row0, _GCH)], rbuf)
                pltpu.sync_copy(rbuf, spm.at[idx_v.at[kk]], add=True)

        @pl.when(c == 0)
        def _():
            scat(e_h)

        @pl.when(c == 1)
        def _():
            scat(w_h)

        plsc.subcore_barrier()

        def writeback(out_h):
            pltpu.sync_copy(spm.at[pl.ds(sid * _ZR, _ZR)],
                            out_h.at[pl.ds(sid * _ZR, _ZR)])

            @pl.when(sid == 15)
            def _():
                pltpu.sync_copy(spm.at[pl.ds(16 * _ZR, _N - 16 * _ZR)],
                                out_h.at[pl.ds(16 * _ZR, _N - 16 * _ZR)])

        @pl.when(c == 0)
        def _():
            writeback(s_out)

        @pl.when(c == 1)
        def _():
            writeback(a_out)

    return k(e_arr, w_arr, idx2, zeros)


def _bn_affine(st, g, be):
    mu = st[0] / _E
    var = st[1] / _E - mu * mu
    scale = g * lax.rsqrt(var + 1e-5)
    shift = be - mu * scale
    return scale, shift


def kernel(x, pos, edge_index, W_lin, W_src, W_dst,
           pW1, pb1, pg1, pbe1, pW2, pb2, pg2, pbe2,
           aW1, ab1, ag1, abe1, aW2, ab2, ag2, abe2,
           up_W, up_b):
    src = jnp.pad(edge_index[0], (0, _EP - _E)).reshape(_GNC, _GCH)
    dst = jnp.pad(edge_index[1], (0, _EP - _E)).reshape(_GNC, _GCH)
    pos16 = jnp.pad(pos, ((0, 0), (0, 13)))
    w1p = jnp.pad(pW1, ((0, 13), (0, 0)))
    wcat = jnp.concatenate([W_lin, W_src, W_dst], axis=1)

    xcat, p_t = _prep(x, pos16, wcat, w1p)
    v = xcat[:, :_C]
    a_src = xcat[:, _C:2 * _C]
    a_dst = xcat[:, 2 * _C:]

    gad, gas, gv, gpd, gps = _sc_gather(a_dst, a_src, v, p_t, dst, src)

    st1 = _pass1(gpd, gps, pb1.reshape(1, _C))
    sc1, sh1 = _bn_affine(st1, pg1, pbe1)
    s1 = jnp.stack([sc1, pb1 * sc1 + sh1])

    h2, st2 = _pass2(gpd, gps, s1, pW2, pb2.reshape(1, _C))
    sc2, sh2 = _bn_affine(st2, pg2, pbe2)
    s2 = jnp.stack([sc2, sh2])

    h3, st3 = _pass3(h2, gad, gas, s2, aW1, ab1.reshape(1, _C))
    sc3, sh3 = _bn_affine(st3, ag1, abe1)
    s3 = jnp.stack([sc3, sh3])

    h4, st4 = _pass4(h3, s3, aW2, ab2.reshape(1, _C))
    sc4, sh4 = _bn_affine(st4, ag2, abe2)
    s4 = jnp.stack([sc4, sh4])

    e, w = _pass5(h4, h2, gv, s4, s2)

    zeros = jnp.zeros((_N, _C), _f32)
    s_sum, acc = _sc_scatter(e, w, dst, zeros)

    return _final(s_sum, acc, x, up_W, up_b.reshape(1, _C))


# speedup vs baseline: 2.4840x; 1.3967x over previous
import functools

import jax
import jax.numpy as jnp
from jax import lax
from jax.experimental import pallas as pl
from jax.experimental.pallas import tpu as pltpu
from jax.experimental.pallas import tpu_sc as plsc

_N = 10000
_E = 320000
_C = 128

_EP = 327680
_B = 4096
_GRID = _EP // _B

_GCH = 128
_GNC = _EP // _GCH
_CPW = _GNC // 32

_SPS = _GNC // 16
_SEGC = 32
_NSEG = _SPS // _SEGC
_ZR = 624

_f32 = jnp.float32


def _prep_body(x_ref, pos_ref, w_ref, w1_ref, ad_ref, as_ref, v_ref, p_ref):
    xw = jnp.dot(x_ref[...], w_ref[...], preferred_element_type=_f32)
    ad_ref[...] = xw[:, :_C]
    as_ref[...] = xw[:, _C:2 * _C]
    v_ref[...] = xw[:, 2 * _C:]
    p_ref[...] = jnp.dot(pos_ref[...], w1_ref[...],
                         preferred_element_type=_f32)


def _prep(x, pos16, wcat, w1p):
    return pl.pallas_call(
        _prep_body,
        grid=(5,),
        in_specs=[pl.BlockSpec((2000, _C), lambda i: (i, 0)),
                  pl.BlockSpec((2000, 16), lambda i: (i, 0)),
                  pl.BlockSpec((_C, 3 * _C), lambda i: (0, 0)),
                  pl.BlockSpec((16, _C), lambda i: (0, 0))],
        out_specs=[pl.BlockSpec((2000, _C), lambda i: (i, 0)),
                   pl.BlockSpec((2000, _C), lambda i: (i, 0)),
                   pl.BlockSpec((2000, _C), lambda i: (i, 0)),
                   pl.BlockSpec((2000, _C), lambda i: (i, 0))],
        out_shape=[jax.ShapeDtypeStruct((_N, _C), _f32) for _ in range(4)],
    )(x, pos16, wcat, w1p)


def _edge_mask():
    rows = lax.broadcasted_iota(jnp.int32, (_B, 1), 0) + pl.program_id(0) * _B
    return (rows < _E).astype(_f32)


def _acc_stats(st_ref, h):
    m = _edge_mask()
    hm = h * m
    blk = jnp.concatenate([jnp.sum(hm, 0, keepdims=True),
                           jnp.sum(hm * h, 0, keepdims=True)])

    @pl.when(pl.program_id(0) == 0)
    def _():
        st_ref[...] = jnp.zeros_like(st_ref)

    st_ref[...] += blk


def _p1_body(gpd_ref, gps_ref, b_ref, st_ref):
    h = gpd_ref[...] - gps_ref[...] + b_ref[...]
    _acc_stats(st_ref, h)


def _pass1(gpd, gps, b1):
    return pl.pallas_call(
        _p1_body,
        grid=(_GRID,),
        in_specs=[pl.BlockSpec((_B, _C), lambda i: (i, 0)),
                  pl.BlockSpec((_B, _C), lambda i: (i, 0)),
                  pl.BlockSpec((1, _C), lambda i: (0, 0))],
        out_specs=pl.BlockSpec((2, _C), lambda i: (0, 0)),
        out_shape=jax.ShapeDtypeStruct((2, _C), _f32),
    )(gpd, gps, b1)


def _p2_body(gpd_ref, gps_ref, s1_ref, w2_ref, b2_ref,
             h2_ref, st_ref):
    r = jnp.maximum((gpd_ref[...] - gps_ref[...]) * s1_ref[0:1]
                    + s1_ref[1:2], 0.0)
    h2 = jnp.dot(r, w2_ref[...], preferred_element_type=_f32) + b2_ref[...]
    h2_ref[...] = h2
    _acc_stats(st_ref, h2)


def _pass2(gpd, gps, s1, w2, b2):
    return pl.pallas_call(
        _p2_body,
        grid=(_GRID,),
        in_specs=[pl.BlockSpec((_B, _C), lambda i: (i, 0)),
                  pl.BlockSpec((_B, _C), lambda i: (i, 0)),
                  pl.BlockSpec((2, _C), lambda i: (0, 0)),
                  pl.BlockSpec((_C, _C), lambda i: (0, 0)),
                  pl.BlockSpec((1, _C), lambda i: (0, 0))],
        out_specs=[pl.BlockSpec((_B, _C), lambda i: (i, 0)),
                   pl.BlockSpec((2, _C), lambda i: (0, 0))],
        out_shape=[jax.ShapeDtypeStruct((_EP, _C), _f32),
                   jax.ShapeDtypeStruct((2, _C), _f32)],
    )(gpd, gps, s1, w2, b2)


def _p3_body(h2_ref, gad_ref, gas_ref, s2_ref, w_ref, b_ref,
             h3_ref, st_ref):
    delta = jnp.maximum(h2_ref[...] * s2_ref[0:1] + s2_ref[1:2], 0.0)
    a0 = gad_ref[...] - gas_ref[...] + delta
    h3 = jnp.dot(a0, w_ref[...], preferred_element_type=_f32) + b_ref[...]
    h3_ref[...] = h3
    _acc_stats(st_ref, h3)


def _pass3(h2, gad, gas, s2, w, b):
    return pl.pallas_call(
        _p3_body,
        grid=(_GRID,),
        in_specs=[pl.BlockSpec((_B, _C), lambda i: (i, 0)),
                  pl.BlockSpec((_B, _C), lambda i: (i, 0)),
                  pl.BlockSpec((_B, _C), lambda i: (i, 0)),
                  pl.BlockSpec((2, _C), lambda i: (0, 0)),
                  pl.BlockSpec((_C, _C), lambda i: (0, 0)),
                  pl.BlockSpec((1, _C), lambda i: (0, 0))],
        out_specs=[pl.BlockSpec((_B, _C), lambda i: (i, 0)),
                   pl.BlockSpec((2, _C), lambda i: (0, 0))],
        out_shape=[jax.ShapeDtypeStruct((_EP, _C), _f32),
                   jax.ShapeDtypeStruct((2, _C), _f32)],
    )(h2, gad, gas, s2, w, b)


def _p4_body(h3_ref, s3_ref, w_ref, b_ref, h4_ref, st_ref):
    s = jnp.maximum(h3_ref[...] * s3_ref[0:1] + s3_ref[1:2], 0.0)
    h4 = jnp.dot(s, w_ref[...], preferred_element_type=_f32) + b_ref[...]
    h4_ref[...] = h4
    _acc_stats(st_ref, h4)


def _pass4(h3, s3, w, b):
    return pl.pallas_call(
        _p4_body,
        grid=(_GRID,),
        in_specs=[pl.BlockSpec((_B, _C), lambda i: (i, 0)),
                  pl.BlockSpec((2, _C), lambda i: (0, 0)),
                  pl.BlockSpec((_C, _C), lambda i: (0, 0)),
                  pl.BlockSpec((1, _C), lambda i: (0, 0))],
        out_specs=[pl.BlockSpec((_B, _C), lambda i: (i, 0)),
                   pl.BlockSpec((2, _C), lambda i: (0, 0))],
        out_shape=[jax.ShapeDtypeStruct((_EP, _C), _f32),
                   jax.ShapeDtypeStruct((2, _C), _f32)],
    )(h3, s3, w, b)


def _p5_body(h4_ref, h2_ref, gv_ref, s4_ref, s2_ref, e_ref, w_ref):
    m = _edge_mask()
    alpha = jnp.maximum(h4_ref[...] * s4_ref[0:1] + s4_ref[1:2], 0.0)
    e = jnp.exp(alpha) * m
    delta = jnp.maximum(h2_ref[...] * s2_ref[0:1] + s2_ref[1:2], 0.0)
    e_ref[...] = e
    w_ref[...] = e * (gv_ref[...] + delta)


def _pass5(h4, h2, gv, s4, s2):
    return pl.pallas_call(
        _p5_body,
        grid=(_GRID,),
        in_specs=[pl.BlockSpec((_B, _C), lambda i: (i, 0)),
                  pl.BlockSpec((_B, _C), lambda i: (i, 0)),
                  pl.BlockSpec((_B, _C), lambda i: (i, 0)),
                  pl.BlockSpec((2, _C), lambda i: (0, 0)),
                  pl.BlockSpec((2, _C), lambda i: (0, 0))],
        out_specs=[pl.BlockSpec((_B, _C), lambda i: (i, 0)),
                   pl.BlockSpec((_B, _C), lambda i: (i, 0))],
        out_shape=[jax.ShapeDtypeStruct((_EP, _C), _f32),
                   jax.ShapeDtypeStruct((_EP, _C), _f32)],
    )(h4, h2, gv, s4, s2)


def _fin_body(s_ref, acc_ref, x_ref, w_ref, b_ref, o_ref):
    o = acc_ref[...] / (s_ref[...] + 1e-16)
    o_ref[...] = jnp.dot(o, w_ref[...],
                         preferred_element_type=_f32) + b_ref[...] + x_ref[...]


def _final(s, acc, x, w, b):
    return pl.pallas_call(
        _fin_body,
        grid=(5,),
        in_specs=[pl.BlockSpec((2000, _C), lambda i: (i, 0)),
                  pl.BlockSpec((2000, _C), lambda i: (i, 0)),
                  pl.BlockSpec((2000, _C), lambda i: (i, 0)),
                  pl.BlockSpec((_C, _C), lambda i: (0, 0)),
                  pl.BlockSpec((1, _C), lambda i: (0, 0))],
        out_specs=pl.BlockSpec((2000, _C), lambda i: (i, 0)),
        out_shape=jax.ShapeDtypeStruct((_N, _C), _f32),
    )(s, acc, x, w, b)


def _gather(tabs, idxs, chunk, nslots):
    ns = len(tabs)
    widths = [t.shape[1] for t in tabs]
    cpw = (_EP // chunk) // 32
    mesh = plsc.VectorSubcoreMesh(core_axis_name="c", subcore_axis_name="s")
    out_type = [jax.ShapeDtypeStruct((_EP, wd), _f32) for wd in widths]
    scr = [pltpu.VMEM((cpw, chunk), jnp.int32) for _ in range(ns)]
    for _ in range(nslots):
        scr += [pltpu.VMEM((chunk, wd), _f32) for wd in widths]
        scr += [pltpu.SemaphoreType.DMA, pltpu.SemaphoreType.DMA]

    sl = ns + 2

    @functools.partial(pl.kernel, mesh=mesh, out_type=out_type,
                       scratch_types=scr)
    def k(*refs):
        t_h = refs[:ns]
        i_h = refs[ns:2 * ns]
        o_h = refs[2 * ns:3 * ns]
        i_v = refs[3 * ns:4 * ns]
        slots = refs[4 * ns:]
        w = lax.axis_index("s") * 2 + lax.axis_index("c")
        base = w * cpw
        for t in range(ns):
            pltpu.sync_copy(i_h[t].at[pl.ds(base, cpw)], i_v[t])

        def issue_g(kk, j):
            bufs, gs = slots[sl * j:sl * j + ns], slots[sl * j + ns]
            for t in range(ns):
                pltpu.async_copy(t_h[t].at[i_v[t].at[kk]], bufs[t], gs)

        def wait_g(j):
            bufs, gs = slots[sl * j:sl * j + ns], slots[sl * j + ns]
            for t in range(ns):
                pltpu.make_async_copy(t_h[t].at[i_v[t].at[0]],
                                      bufs[t], gs).wait()

        def issue_w(kk, j):
            bufs, ws = slots[sl * j:sl * j + ns], slots[sl * j + ns + 1]
            row0 = (base + kk) * chunk
            for t in range(ns):
                pltpu.make_async_copy(
                    bufs[t], o_h[t].at[pl.ds(row0, chunk)], ws).start()

        def wait_w(j):
            bufs, ws = slots[sl * j:sl * j + ns], slots[sl * j + ns + 1]
            for t in range(ns):
                pltpu.make_async_copy(
                    bufs[t], o_h[t].at[pl.ds(0, chunk)], ws).wait()

        for j in range(nslots):
            issue_g(j, j)

        nloop = -(-cpw // nslots)

        @pl.loop(0, nloop)
        def _(i):
            for j in range(nslots):
                kk = i * nslots + j

                @pl.when(kk < cpw)
                def _():
                    wait_g(j)
                    issue_w(kk, j)

                    @pl.when(kk + nslots < cpw)
                    def _():
                        wait_w(j)
                        issue_g(kk + nslots, j)

        for j in range(nslots):
            wait_w(j)

    return k(*tabs, *idxs)


def _sc_scatter(e_arr, w_arr, idx2, zeros):
    mesh = plsc.VectorSubcoreMesh(core_axis_name="c", subcore_axis_name="s")
    out_type = [jax.ShapeDtypeStruct((_N, _C), _f32),
                jax.ShapeDtypeStruct((_N, _C), _f32)]

    nslots = 2
    scr = [pltpu.VMEM_SHARED((_N, _C), _f32),
           pltpu.VMEM((_SEGC, _GCH), jnp.int32)]
    for _ in range(nslots):
        scr += [pltpu.VMEM((_GCH, _C), _f32),
                pltpu.SemaphoreType.DMA,
                pltpu.SemaphoreType.DMA]

    @functools.partial(
        pl.kernel, mesh=mesh, out_type=out_type, scratch_types=scr)
    def k(e_h, w_h, idx_h, z_h, s_out, a_out, spm, idx_v, *slots):
        c = lax.axis_index("c")
        sid = lax.axis_index("s")
        pltpu.sync_copy(z_h.at[pl.ds(sid * _ZR, _ZR)],
                        spm.at[pl.ds(sid * _ZR, _ZR)])

        @pl.when(sid == 15)
        def _():
            pltpu.sync_copy(z_h.at[pl.ds(16 * _ZR, _N - 16 * _ZR)],
                            spm.at[pl.ds(16 * _ZR, _N - 16 * _ZR)])

        plsc.subcore_barrier()

        def scat(src_h):
            def issue_l(kk, j):
                b, ls, _ = slots[3 * j:3 * j + 3]
                row0 = (sid * _SPS + kk) * _GCH
                pltpu.async_copy(src_h.at[pl.ds(row0, _GCH)], b, ls)

            def wait_l(j):
                b, ls, _ = slots[3 * j:3 * j + 3]
                pltpu.make_async_copy(src_h.at[pl.ds(0, _GCH)], b, ls).wait()

            def issue_s(q, j):
                b, _, ss = slots[3 * j:3 * j + 3]
                pltpu.async_copy(b, spm.at[idx_v.at[q]], ss, add=True)

            def wait_s(j):
                b, _, ss = slots[3 * j:3 * j + 3]
                pltpu.make_async_copy(b, spm.at[idx_v.at[0]], ss).wait()

            for seg in range(_NSEG):
                pltpu.sync_copy(
                    idx_h.at[pl.ds(sid * _SPS + seg * _SEGC, _SEGC)], idx_v)
                for j in range(nslots):
                    issue_l(seg * _SEGC + j, j)

                @pl.loop(0, _SEGC // nslots)
                def _(i):
                    for j in range(nslots):
                        q = i * nslots + j
                        kk = seg * _SEGC + q
                        wait_l(j)
                        issue_s(q, j)

                        @pl.when(q + nslots < _SEGC)
                        def _():
                            wait_s(j)
                            issue_l(kk + nslots, j)

                for j in range(nslots):
                    wait_s(j)

        @pl.when(c == 0)
        def _():
            scat(e_h)

        @pl.when(c == 1)
        def _():
            scat(w_h)

        plsc.subcore_barrier()

        def writeback(out_h):
            pltpu.sync_copy(spm.at[pl.ds(sid * _ZR, _ZR)],
                            out_h.at[pl.ds(sid * _ZR, _ZR)])

            @pl.when(sid == 15)
            def _():
                pltpu.sync_copy(spm.at[pl.ds(16 * _ZR, _N - 16 * _ZR)],
                                out_h.at[pl.ds(16 * _ZR, _N - 16 * _ZR)])

        @pl.when(c == 0)
        def _():
            writeback(s_out)

        @pl.when(c == 1)
        def _():
            writeback(a_out)

    return k(e_arr, w_arr, idx2, zeros)


def _bn_affine(st, g, be):
    mu = st[0] / _E
    var = st[1] / _E - mu * mu
    scale = g * lax.rsqrt(var + 1e-5)
    shift = be - mu * scale
    return scale, shift


def kernel(x, pos, edge_index, W_lin, W_src, W_dst,
           pW1, pb1, pg1, pbe1, pW2, pb2, pg2, pbe2,
           aW1, ab1, ag1, abe1, aW2, ab2, ag2, abe2,
           up_W, up_b):
    src_p = jnp.pad(edge_index[0], (0, _EP - _E))
    dst_p = jnp.pad(edge_index[1], (0, _EP - _E))
    src128 = src_p.reshape(_GNC, _GCH)
    dst128 = dst_p.reshape(_GNC, _GCH)
    pos16 = jnp.pad(pos, ((0, 0), (0, 13)))
    w1p = jnp.pad(pW1, ((0, 13), (0, 0)))
    wcat = jnp.concatenate([W_dst, W_src, W_lin], axis=1)

    a_dst, a_srcT, v_t, p_t = _prep(x, pos16, wcat, w1p)

    gpd, gps = _gather([p_t, p_t], [dst128, src128], _GCH, 2)
    gad, gas = _gather([a_dst, a_srcT], [dst128, src128], _GCH, 2)
    gv, = _gather([v_t], [src128], _GCH, 3)

    st1 = _pass1(gpd, gps, pb1.reshape(1, _C))
    sc1, sh1 = _bn_affine(st1, pg1, pbe1)
    s1 = jnp.stack([sc1, pb1 * sc1 + sh1])

    h2, st2 = _pass2(gpd, gps, s1, pW2, pb2.reshape(1, _C))
    sc2, sh2 = _bn_affine(st2, pg2, pbe2)
    s2 = jnp.stack([sc2, sh2])

    h3, st3 = _pass3(h2, gad, gas, s2, aW1, ab1.reshape(1, _C))
    sc3, sh3 = _bn_affine(st3, ag1, abe1)
    s3 = jnp.stack([sc3, sh3])

    h4, st4 = _pass4(h3, s3, aW2, ab2.reshape(1, _C))
    sc4, sh4 = _bn_affine(st4, ag2, abe2)
    s4 = jnp.stack([sc4, sh4])

    e, w = _pass5(h4, h2, gv, s4, s2)

    zeros = jnp.zeros((_N, _C), _f32)
    s_sum, acc = _sc_scatter(e, w, dst128, zeros)

    return _final(s_sum, acc, x, up_W, up_b.reshape(1, _C))


# speedup vs baseline: 2.9819x; 1.2004x over previous
import functools

import jax
import jax.numpy as jnp
from jax import lax
from jax.experimental import pallas as pl
from jax.experimental.pallas import tpu as pltpu
from jax.experimental.pallas import tpu_sc as plsc

_N = 10000
_E = 320000
_C = 128

_EP = 327680
_B = 4096
_GRID = _EP // _B

_GCH = 128
_GNC = _EP // _GCH
_CPW = _GNC // 32
_GC2 = 64

_SPS = _GNC // 16
_SEGC = 32
_NSEG = _SPS // _SEGC
_ZR = 624

_f32 = jnp.float32
_bf16 = jnp.bfloat16


def _prep_body(x_ref, pos_ref, w_ref, w1_ref, td_ref, ts_ref):
    xw = jnp.dot(x_ref[...], w_ref[...], preferred_element_type=_f32)
    p = jnp.dot(pos_ref[...], w1_ref[...], preferred_element_type=_f32)
    td_ref[...] = jnp.concatenate([xw[:, :_C], p], 1)
    ts_ref[...] = jnp.concatenate(
        [xw[:, _C:2 * _C], p, xw[:, 2 * _C:]], 1)


def _prep(x, pos16, wcat, w1p):
    return pl.pallas_call(
        _prep_body,
        grid=(5,),
        in_specs=[pl.BlockSpec((2000, _C), lambda i: (i, 0)),
                  pl.BlockSpec((2000, 16), lambda i: (i, 0)),
                  pl.BlockSpec((_C, 3 * _C), lambda i: (0, 0)),
                  pl.BlockSpec((16, _C), lambda i: (0, 0))],
        out_specs=[pl.BlockSpec((2000, 2 * _C), lambda i: (i, 0)),
                   pl.BlockSpec((2000, 3 * _C), lambda i: (i, 0))],
        out_shape=[jax.ShapeDtypeStruct((_N, 2 * _C), _f32),
                   jax.ShapeDtypeStruct((_N, 3 * _C), _f32)],
    )(x, pos16, wcat, w1p)


def _edge_mask():
    rows = lax.broadcasted_iota(jnp.int32, (_B, 1), 0) + pl.program_id(0) * _B
    return (rows < _E).astype(_f32)


def _acc_stats(st_ref, h):
    m = _edge_mask()
    hm = h * m
    blk = jnp.concatenate([jnp.sum(hm, 0, keepdims=True),
                           jnp.sum(hm * h, 0, keepdims=True)])

    @pl.when(pl.program_id(0) == 0)
    def _():
        st_ref[...] = jnp.zeros_like(st_ref)

    st_ref[...] += blk


def _p1_body(gpd_ref, gps_ref, b_ref, st_ref):
    h = gpd_ref[...] - gps_ref[...] + b_ref[...]
    _acc_stats(st_ref, h)


def _pass1(gpd, gps, b1):
    return pl.pallas_call(
        _p1_body,
        grid=(_GRID,),
        in_specs=[pl.BlockSpec((_B, _C), lambda i: (i, 1)),
                  pl.BlockSpec((_B, _C), lambda i: (i, 1)),
                  pl.BlockSpec((1, _C), lambda i: (0, 0))],
        out_specs=pl.BlockSpec((2, _C), lambda i: (0, 0)),
        out_shape=jax.ShapeDtypeStruct((2, _C), _f32),
    )(gpd, gps, b1)


def _p2_body(gpd_ref, gps_ref, s1_ref, w2_ref, b2_ref,
             h2_ref, st_ref):
    r = jnp.maximum((gpd_ref[...] - gps_ref[...])
                    * s1_ref[0:1] + s1_ref[1:2], 0.0)
    h2 = jnp.dot(r, w2_ref[...], preferred_element_type=_f32) + b2_ref[...]
    h2_ref[...] = h2
    _acc_stats(st_ref, h2)


def _pass2(gpd, gps, s1, w2, b2):
    return pl.pallas_call(
        _p2_body,
        grid=(_GRID,),
        in_specs=[pl.BlockSpec((_B, _C), lambda i: (i, 1)),
                  pl.BlockSpec((_B, _C), lambda i: (i, 1)),
                  pl.BlockSpec((2, _C), lambda i: (0, 0)),
                  pl.BlockSpec((_C, _C), lambda i: (0, 0)),
                  pl.BlockSpec((1, _C), lambda i: (0, 0))],
        out_specs=[pl.BlockSpec((_B, _C), lambda i: (i, 0)),
                   pl.BlockSpec((2, _C), lambda i: (0, 0))],
        out_shape=[jax.ShapeDtypeStruct((_EP, _C), _f32),
                   jax.ShapeDtypeStruct((2, _C), _f32)],
    )(gpd, gps, s1, w2, b2)


def _p3_body(h2_ref, gad_ref, gas_ref, s2_ref, w_ref, b_ref,
             h3_ref, st_ref):
    delta = jnp.maximum(h2_ref[...] * s2_ref[0:1] + s2_ref[1:2], 0.0)
    a0 = gad_ref[...] - gas_ref[...] + delta
    h3 = jnp.dot(a0, w_ref[...], preferred_element_type=_f32) + b_ref[...]
    h3_ref[...] = h3
    _acc_stats(st_ref, h3)


def _pass3(h2, gad, gas, s2, w, b):
    return pl.pallas_call(
        _p3_body,
        grid=(_GRID,),
        in_specs=[pl.BlockSpec((_B, _C), lambda i: (i, 0)),
                  pl.BlockSpec((_B, _C), lambda i: (i, 0)),
                  pl.BlockSpec((_B, _C), lambda i: (i, 0)),
                  pl.BlockSpec((2, _C), lambda i: (0, 0)),
                  pl.BlockSpec((_C, _C), lambda i: (0, 0)),
                  pl.BlockSpec((1, _C), lambda i: (0, 0))],
        out_specs=[pl.BlockSpec((_B, _C), lambda i: (i, 0)),
                   pl.BlockSpec((2, _C), lambda i: (0, 0))],
        out_shape=[jax.ShapeDtypeStruct((_EP, _C), _f32),
                   jax.ShapeDtypeStruct((2, _C), _f32)],
    )(h2, gad, gas, s2, w, b)


def _p4_body(h3_ref, s3_ref, w_ref, b_ref, h4_ref, st_ref):
    s = jnp.maximum(h3_ref[...] * s3_ref[0:1] + s3_ref[1:2], 0.0)
    h4 = jnp.dot(s, w_ref[...], preferred_element_type=_f32) + b_ref[...]
    h4_ref[...] = h4
    _acc_stats(st_ref, h4)


def _pass4(h3, s3, w, b):
    return pl.pallas_call(
        _p4_body,
        grid=(_GRID,),
        in_specs=[pl.BlockSpec((_B, _C), lambda i: (i, 0)),
                  pl.BlockSpec((2, _C), lambda i: (0, 0)),
                  pl.BlockSpec((_C, _C), lambda i: (0, 0)),
                  pl.BlockSpec((1, _C), lambda i: (0, 0))],
        out_specs=[pl.BlockSpec((_B, _C), lambda i: (i, 0)),
                   pl.BlockSpec((2, _C), lambda i: (0, 0))],
        out_shape=[jax.ShapeDtypeStruct((_EP, _C), _f32),
                   jax.ShapeDtypeStruct((2, _C), _f32)],
    )(h3, s3, w, b)


def _p5_body(h4_ref, h2_ref, gv_ref, s4_ref, s2_ref, e_ref, w_ref):
    m = _edge_mask()
    alpha = jnp.maximum(h4_ref[...] * s4_ref[0:1] + s4_ref[1:2], 0.0)
    e = jnp.exp(alpha) * m
    delta = jnp.maximum(h2_ref[...] * s2_ref[0:1] + s2_ref[1:2], 0.0)
    e_ref[...] = e
    w_ref[...] = e * (gv_ref[...] + delta)


def _pass5(h4, h2, gv, s4, s2):
    return pl.pallas_call(
        _p5_body,
        grid=(_GRID,),
        in_specs=[pl.BlockSpec((_B, _C), lambda i: (i, 0)),
                  pl.BlockSpec((_B, _C), lambda i: (i, 0)),
                  pl.BlockSpec((_B, _C), lambda i: (i, 2)),
                  pl.BlockSpec((2, _C), lambda i: (0, 0)),
                  pl.BlockSpec((2, _C), lambda i: (0, 0))],
        out_specs=[pl.BlockSpec((_B, _C), lambda i: (i, 0)),
                   pl.BlockSpec((_B, _C), lambda i: (i, 0))],
        out_shape=[jax.ShapeDtypeStruct((_EP, _C), _f32),
                   jax.ShapeDtypeStruct((_EP, _C), _f32)],
    )(h4, h2, gv, s4, s2)


def _fin_body(s_ref, acc_ref, x_ref, w_ref, b_ref, o_ref):
    o = acc_ref[...] / (s_ref[...] + 1e-16)
    o_ref[...] = jnp.dot(o, w_ref[...],
                         preferred_element_type=_f32) + b_ref[...] + x_ref[...]


def _final(s, acc, x, w, b):
    return pl.pallas_call(
        _fin_body,
        grid=(5,),
        in_specs=[pl.BlockSpec((2000, _C), lambda i: (i, 0)),
                  pl.BlockSpec((2000, _C), lambda i: (i, 0)),
                  pl.BlockSpec((2000, _C), lambda i: (i, 0)),
                  pl.BlockSpec((_C, _C), lambda i: (0, 0)),
                  pl.BlockSpec((1, _C), lambda i: (0, 0))],
        out_specs=pl.BlockSpec((2000, _C), lambda i: (i, 0)),
        out_shape=jax.ShapeDtypeStruct((_N, _C), _f32),
    )(s, acc, x, w, b)


def _gather(tabs, idxs, chunk, nslots):
    ns = len(tabs)
    rows = [t.shape[1:] for t in tabs]
    dts = [t.dtype for t in tabs]
    cpw = (_EP // chunk) // 32
    mesh = plsc.VectorSubcoreMesh(core_axis_name="c", subcore_axis_name="s")
    out_type = [jax.ShapeDtypeStruct((_EP,) + r, d) for r, d in zip(rows, dts)]
    scr = [pltpu.VMEM((cpw, chunk), jnp.int32) for _ in range(ns)]
    for _ in range(nslots):
        scr += [pltpu.VMEM((chunk,) + r, d) for r, d in zip(rows, dts)]
        scr += [pltpu.SemaphoreType.DMA, pltpu.SemaphoreType.DMA]

    sl = ns + 2

    @functools.partial(pl.kernel, mesh=mesh, out_type=out_type,
                       scratch_types=scr)
    def k(*refs):
        t_h = refs[:ns]
        i_h = refs[ns:2 * ns]
        o_h = refs[2 * ns:3 * ns]
        i_v = refs[3 * ns:4 * ns]
        slots = refs[4 * ns:]
        w = lax.axis_index("s") * 2 + lax.axis_index("c")
        base = w * cpw
        for t in range(ns):
            pltpu.sync_copy(i_h[t].at[pl.ds(base, cpw)], i_v[t])

        def issue_g(kk, j):
            bufs, gs = slots[sl * j:sl * j + ns], slots[sl * j + ns]
            for t in range(ns):
                pltpu.async_copy(t_h[t].at[i_v[t].at[kk]], bufs[t], gs)

        def wait_g(j):
            bufs, gs = slots[sl * j:sl * j + ns], slots[sl * j + ns]
            for t in range(ns):
                pltpu.make_async_copy(t_h[t].at[i_v[t].at[0]],
                                      bufs[t], gs).wait()

        def issue_w(kk, j):
            bufs, ws = slots[sl * j:sl * j + ns], slots[sl * j + ns + 1]
            row0 = (base + kk) * chunk
            for t in range(ns):
                pltpu.make_async_copy(
                    bufs[t], o_h[t].at[pl.ds(row0, chunk)], ws).start()

        def wait_w(j):
            bufs, ws = slots[sl * j:sl * j + ns], slots[sl * j + ns + 1]
            for t in range(ns):
                pltpu.make_async_copy(
                    bufs[t], o_h[t].at[pl.ds(0, chunk)], ws).wait()

        for j in range(nslots):
            issue_g(j, j)

        nloop = -(-cpw // nslots)

        @pl.loop(0, nloop)
        def _(i):
            for j in range(nslots):
                kk = i * nslots + j

                @pl.when(kk < cpw)
                def _():
                    wait_g(j)
                    issue_w(kk, j)

                    @pl.when(kk + nslots < cpw)
                    def _():
                        wait_w(j)
                        issue_g(kk + nslots, j)

        for j in range(nslots):
            wait_w(j)

    return k(*tabs, *idxs)


def _sc_scatter(e_arr, w_arr, idx2, zeros):
    mesh = plsc.VectorSubcoreMesh(core_axis_name="c", subcore_axis_name="s")
    out_type = [jax.ShapeDtypeStruct((_N, _C), _f32),
                jax.ShapeDtypeStruct((_N, _C), _f32)]

    nslots = 2
    scr = [pltpu.VMEM_SHARED((_N, _C), _f32),
           pltpu.VMEM((_SEGC, _GCH), jnp.int32)]
    for _ in range(nslots):
        scr += [pltpu.VMEM((_GCH, _C), _f32),
                pltpu.SemaphoreType.DMA,
                pltpu.SemaphoreType.DMA]

    @functools.partial(
        pl.kernel, mesh=mesh, out_type=out_type, scratch_types=scr)
    def k(e_h, w_h, idx_h, z_h, s_out, a_out, spm, idx_v, *slots):
        c = lax.axis_index("c")
        sid = lax.axis_index("s")
        pltpu.sync_copy(z_h.at[pl.ds(sid * _ZR, _ZR)],
                        spm.at[pl.ds(sid * _ZR, _ZR)])

        @pl.when(sid == 15)
        def _():
            pltpu.sync_copy(z_h.at[pl.ds(16 * _ZR, _N - 16 * _ZR)],
                            spm.at[pl.ds(16 * _ZR, _N - 16 * _ZR)])

        plsc.subcore_barrier()

        def scat(src_h):
            def issue_l(kk, j):
                b, ls, _ = slots[3 * j:3 * j + 3]
                row0 = (sid * _SPS + kk) * _GCH
                pltpu.async_copy(src_h.at[pl.ds(row0, _GCH)], b, ls)

            def wait_l(j):
                b, ls, _ = slots[3 * j:3 * j + 3]
                pltpu.make_async_copy(src_h.at[pl.ds(0, _GCH)], b, ls).wait()

            def issue_s(q, j):
                b, _, ss = slots[3 * j:3 * j + 3]
                pltpu.async_copy(b, spm.at[idx_v.at[q]], ss, add=True)

            def wait_s(j):
                b, _, ss = slots[3 * j:3 * j + 3]
                pltpu.make_async_copy(b, spm.at[idx_v.at[0]], ss).wait()

            for seg in range(_NSEG):
                pltpu.sync_copy(
                    idx_h.at[pl.ds(sid * _SPS + seg * _SEGC, _SEGC)], idx_v)
                for j in range(nslots):
                    issue_l(seg * _SEGC + j, j)

                @pl.loop(0, _SEGC // nslots)
                def _(i):
                    for j in range(nslots):
                        q = i * nslots + j
                        kk = seg * _SEGC + q
                        wait_l(j)
                        issue_s(q, j)

                        @pl.when(q + nslots < _SEGC)
                        def _():
                            wait_s(j)
                            issue_l(kk + nslots, j)

                for j in range(nslots):
                    wait_s(j)

        @pl.when(c == 0)
        def _():
            scat(e_h)

        @pl.when(c == 1)
        def _():
            scat(w_h)

        plsc.subcore_barrier()

        def writeback(out_h):
            pltpu.sync_copy(spm.at[pl.ds(sid * _ZR, _ZR)],
                            out_h.at[pl.ds(sid * _ZR, _ZR)])

            @pl.when(sid == 15)
            def _():
                pltpu.sync_copy(spm.at[pl.ds(16 * _ZR, _N - 16 * _ZR)],
                                out_h.at[pl.ds(16 * _ZR, _N - 16 * _ZR)])

        @pl.when(c == 0)
        def _():
            writeback(s_out)

        @pl.when(c == 1)
        def _():
            writeback(a_out)

    return k(e_arr, w_arr, idx2, zeros)


def _bn_affine(st, g, be):
    mu = st[0] / _E
    var = st[1] / _E - mu * mu
    scale = g * lax.rsqrt(var + 1e-5)
    shift = be - mu * scale
    return scale, shift


def kernel(x, pos, edge_index, W_lin, W_src, W_dst,
           pW1, pb1, pg1, pbe1, pW2, pb2, pg2, pbe2,
           aW1, ab1, ag1, abe1, aW2, ab2, ag2, abe2,
           up_W, up_b):
    src_p = jnp.pad(edge_index[0], (0, _EP - _E))
    dst_p = jnp.pad(edge_index[1], (0, _EP - _E))
    src128 = src_p.reshape(_GNC, _GCH)
    dst128 = dst_p.reshape(_GNC, _GCH)
    src64 = src_p.reshape(_EP // _GC2, _GC2)
    dst64 = dst_p.reshape(_EP // _GC2, _GC2)
    pos16 = jnp.pad(pos, ((0, 0), (0, 13)))
    w1p = jnp.pad(pW1, ((0, 13), (0, 0)))
    wcat = jnp.concatenate([W_dst, W_src, W_lin], axis=1)

    td, ts = _prep(x, pos16, wcat, w1p)

    gd, = _gather([td], [dst64], _GC2, 3)
    gs, = _gather([ts], [src64], _GC2, 3)

    st1 = _pass1(gd, gs, pb1.reshape(1, _C))
    sc1, sh1 = _bn_affine(st1, pg1, pbe1)
    s1 = jnp.stack([sc1, pb1 * sc1 + sh1])

    h2, st2 = _pass2(gd, gs, s1, pW2, pb2.reshape(1, _C))
    sc2, sh2 = _bn_affine(st2, pg2, pbe2)
    s2 = jnp.stack([sc2, sh2])

    h3, st3 = _pass3(h2, gd, gs, s2, aW1, ab1.reshape(1, _C))
    sc3, sh3 = _bn_affine(st3, ag1, abe1)
    s3 = jnp.stack([sc3, sh3])

    h4, st4 = _pass4(h3, s3, aW2, ab2.reshape(1, _C))
    sc4, sh4 = _bn_affine(st4, ag2, abe2)
    s4 = jnp.stack([sc4, sh4])

    e, w = _pass5(h4, h2, gs, s4, s2)

    zeros = jnp.zeros((_N, _C), _f32)
    s_sum, acc = _sc_scatter(e, w, dst128, zeros)

    return _final(s_sum, acc, x, up_W, up_b.reshape(1, _C))


# speedup vs baseline: 3.3366x; 1.1190x over previous
import functools

import jax
import jax.numpy as jnp
from jax import lax
from jax.experimental import pallas as pl
from jax.experimental.pallas import tpu as pltpu
from jax.experimental.pallas import tpu_sc as plsc

_N = 10000
_E = 320000
_C = 128

_EP = 327680
_B = 4096
_GRID = _EP // _B

_GCH = 128
_GNC = _EP // _GCH
_CPW = _GNC // 32
_GC2 = 64

_SPS = _GNC // 16
_SEGC = 32
_NSEG = _SPS // _SEGC
_ZR = 624

_f32 = jnp.float32
_bf16 = jnp.bfloat16
_u32 = jnp.uint32


def _bf16_bits(a):
    r = jax.lax.bitcast_convert_type(a, _u32)
    return (r + jnp.uint32(0x7FFF) + ((r >> 16) & jnp.uint32(1))) >> 16


def _pack2(a, b):
    return _bf16_bits(a) | (_bf16_bits(b) << 16)


def _lo_f32(w):
    return jax.lax.bitcast_convert_type(w << 16, _f32)


def _hi_f32(w):
    return jax.lax.bitcast_convert_type(w & jnp.uint32(0xFFFF0000), _f32)


def _prep_body(x_ref, pos_ref, w_ref, w1_ref, td_ref, ts_ref):
    xw = jnp.dot(x_ref[...], w_ref[...], preferred_element_type=_f32)
    p = jnp.dot(pos_ref[...], w1_ref[...], preferred_element_type=_f32)
    td_ref[...] = _pack2(xw[:, :_C], p)
    ts_ref[...] = jnp.concatenate(
        [_pack2(xw[:, _C:2 * _C], p),
         jax.lax.bitcast_convert_type(xw[:, 2 * _C:], _u32)], 1)


def _prep(x, pos16, wcat, w1p):
    return pl.pallas_call(
        _prep_body,
        grid=(5,),
        in_specs=[pl.BlockSpec((2000, _C), lambda i: (i, 0)),
                  pl.BlockSpec((2000, 16), lambda i: (i, 0)),
                  pl.BlockSpec((_C, 3 * _C), lambda i: (0, 0)),
                  pl.BlockSpec((16, _C), lambda i: (0, 0))],
        out_specs=[pl.BlockSpec((2000, _C), lambda i: (i, 0)),
                   pl.BlockSpec((2000, 2 * _C), lambda i: (i, 0))],
        out_shape=[jax.ShapeDtypeStruct((_N, _C), _u32),
                   jax.ShapeDtypeStruct((_N, 2 * _C), _u32)],
    )(x, pos16, wcat, w1p)


def _edge_mask():
    rows = lax.broadcasted_iota(jnp.int32, (_B, 1), 0) + pl.program_id(0) * _B
    return (rows < _E).astype(_f32)


def _acc_stats(st_ref, h):
    m = _edge_mask()
    hm = h * m
    blk = jnp.concatenate([jnp.sum(hm, 0, keepdims=True),
                           jnp.sum(hm * h, 0, keepdims=True)])

    @pl.when(pl.program_id(0) == 0)
    def _():
        st_ref[...] = jnp.zeros_like(st_ref)

    st_ref[...] += blk


def _p1_body(gpd_ref, gps_ref, b_ref, st_ref):
    h = _hi_f32(gpd_ref[...]) - _hi_f32(gps_ref[...]) + b_ref[...]
    _acc_stats(st_ref, h)


def _pass1(gpd, gps, b1):
    return pl.pallas_call(
        _p1_body,
        grid=(_GRID,),
        in_specs=[pl.BlockSpec((_B, _C), lambda i: (i, 0)),
                  pl.BlockSpec((_B, _C), lambda i: (i, 0)),
                  pl.BlockSpec((1, _C), lambda i: (0, 0))],
        out_specs=pl.BlockSpec((2, _C), lambda i: (0, 0)),
        out_shape=jax.ShapeDtypeStruct((2, _C), _f32),
    )(gpd, gps, b1)


def _p2_body(gpd_ref, gps_ref, s1_ref, w2_ref, b2_ref,
             h2_ref, st_ref):
    r = jnp.maximum((_hi_f32(gpd_ref[...]) - _hi_f32(gps_ref[...]))
                    * s1_ref[0:1] + s1_ref[1:2], 0.0)
    h2 = jnp.dot(r, w2_ref[...], preferred_element_type=_f32) + b2_ref[...]
    h2_ref[...] = h2
    _acc_stats(st_ref, h2)


def _pass2(gpd, gps, s1, w2, b2):
    return pl.pallas_call(
        _p2_body,
        grid=(_GRID,),
        in_specs=[pl.BlockSpec((_B, _C), lambda i: (i, 0)),
                  pl.BlockSpec((_B, _C), lambda i: (i, 0)),
                  pl.BlockSpec((2, _C), lambda i: (0, 0)),
                  pl.BlockSpec((_C, _C), lambda i: (0, 0)),
                  pl.BlockSpec((1, _C), lambda i: (0, 0))],
        out_specs=[pl.BlockSpec((_B, _C), lambda i: (i, 0)),
                   pl.BlockSpec((2, _C), lambda i: (0, 0))],
        out_shape=[jax.ShapeDtypeStruct((_EP, _C), _f32),
                   jax.ShapeDtypeStruct((2, _C), _f32)],
    )(gpd, gps, s1, w2, b2)


def _p3_body(h2_ref, gad_ref, gas_ref, s2_ref, w_ref, b_ref,
             h3_ref, st_ref):
    delta = jnp.maximum(h2_ref[...] * s2_ref[0:1] + s2_ref[1:2], 0.0)
    a0 = _lo_f32(gad_ref[...]) - _lo_f32(gas_ref[...]) + delta
    h3 = jnp.dot(a0, w_ref[...], preferred_element_type=_f32) + b_ref[...]
    h3_ref[...] = h3
    _acc_stats(st_ref, h3)


def _pass3(h2, gad, gas, s2, w, b):
    return pl.pallas_call(
        _p3_body,
        grid=(_GRID,),
        in_specs=[pl.BlockSpec((_B, _C), lambda i: (i, 0)),
                  pl.BlockSpec((_B, _C), lambda i: (i, 0)),
                  pl.BlockSpec((_B, _C), lambda i: (i, 0)),
                  pl.BlockSpec((2, _C), lambda i: (0, 0)),
                  pl.BlockSpec((_C, _C), lambda i: (0, 0)),
                  pl.BlockSpec((1, _C), lambda i: (0, 0))],
        out_specs=[pl.BlockSpec((_B, _C), lambda i: (i, 0)),
                   pl.BlockSpec((2, _C), lambda i: (0, 0))],
        out_shape=[jax.ShapeDtypeStruct((_EP, _C), _f32),
                   jax.ShapeDtypeStruct((2, _C), _f32)],
    )(h2, gad, gas, s2, w, b)


def _p4_body(h3_ref, s3_ref, w_ref, b_ref, h4_ref, st_ref):
    s = jnp.maximum(h3_ref[...] * s3_ref[0:1] + s3_ref[1:2], 0.0)
    h4 = jnp.dot(s, w_ref[...], preferred_element_type=_f32) + b_ref[...]
    h4_ref[...] = h4
    _acc_stats(st_ref, h4)


def _pass4(h3, s3, w, b):
    return pl.pallas_call(
        _p4_body,
        grid=(_GRID,),
        in_specs=[pl.BlockSpec((_B, _C), lambda i: (i, 0)),
                  pl.BlockSpec((2, _C), lambda i: (0, 0)),
                  pl.BlockSpec((_C, _C), lambda i: (0, 0)),
                  pl.BlockSpec((1, _C), lambda i: (0, 0))],
        out_specs=[pl.BlockSpec((_B, _C), lambda i: (i, 0)),
                   pl.BlockSpec((2, _C), lambda i: (0, 0))],
        out_shape=[jax.ShapeDtypeStruct((_EP, _C), _f32),
                   jax.ShapeDtypeStruct((2, _C), _f32)],
    )(h3, s3, w, b)


def _p5_body(h4_ref, h2_ref, gv_ref, s4_ref, s2_ref, e_ref, w_ref):
    m = _edge_mask()
    alpha = jnp.maximum(h4_ref[...] * s4_ref[0:1] + s4_ref[1:2], 0.0)
    e = jnp.exp(alpha) * m
    delta = jnp.maximum(h2_ref[...] * s2_ref[0:1] + s2_ref[1:2], 0.0)
    e_ref[...] = e
    w_ref[...] = e * (jax.lax.bitcast_convert_type(gv_ref[...], _f32)
                      + delta)


def _pass5(h4, h2, gv, s4, s2):
    return pl.pallas_call(
        _p5_body,
        grid=(_GRID,),
        in_specs=[pl.BlockSpec((_B, _C), lambda i: (i, 0)),
                  pl.BlockSpec((_B, _C), lambda i: (i, 0)),
                  pl.BlockSpec((_B, _C), lambda i: (i, 1)),
                  pl.BlockSpec((2, _C), lambda i: (0, 0)),
                  pl.BlockSpec((2, _C), lambda i: (0, 0))],
        out_specs=[pl.BlockSpec((_B, _C), lambda i: (i, 0)),
                   pl.BlockSpec((_B, _C), lambda i: (i, 0))],
        out_shape=[jax.ShapeDtypeStruct((_EP, _C), _f32),
                   jax.ShapeDtypeStruct((_EP, _C), _f32)],
    )(h4, h2, gv, s4, s2)


def _fin_body(s_ref, acc_ref, x_ref, w_ref, b_ref, o_ref):
    o = acc_ref[...] / (s_ref[...] + 1e-16)
    o_ref[...] = jnp.dot(o, w_ref[...],
                         preferred_element_type=_f32) + b_ref[...] + x_ref[...]


def _final(s, acc, x, w, b):
    return pl.pallas_call(
        _fin_body,
        grid=(5,),
        in_specs=[pl.BlockSpec((2000, _C), lambda i: (i, 0)),
                  pl.BlockSpec((2000, _C), lambda i: (i, 0)),
                  pl.BlockSpec((2000, _C), lambda i: (i, 0)),
                  pl.BlockSpec((_C, _C), lambda i: (0, 0)),
                  pl.BlockSpec((1, _C), lambda i: (0, 0))],
        out_specs=pl.BlockSpec((2000, _C), lambda i: (i, 0)),
        out_shape=jax.ShapeDtypeStruct((_N, _C), _f32),
    )(s, acc, x, w, b)


def _gather(tabs, idxs, chunk, nslots):
    ns = len(tabs)
    rows = [t.shape[1:] for t in tabs]
    dts = [t.dtype for t in tabs]
    cpw = (_EP // chunk) // 32
    mesh = plsc.VectorSubcoreMesh(core_axis_name="c", subcore_axis_name="s")
    out_type = [jax.ShapeDtypeStruct((_EP,) + r, d) for r, d in zip(rows, dts)]
    scr = [pltpu.VMEM((cpw, chunk), jnp.int32) for _ in range(ns)]
    for _ in range(nslots):
        scr += [pltpu.VMEM((chunk,) + r, d) for r, d in zip(rows, dts)]
        scr += [pltpu.SemaphoreType.DMA, pltpu.SemaphoreType.DMA]

    sl = ns + 2

    @functools.partial(pl.kernel, mesh=mesh, out_type=out_type,
                       scratch_types=scr)
    def k(*refs):
        t_h = refs[:ns]
        i_h = refs[ns:2 * ns]
        o_h = refs[2 * ns:3 * ns]
        i_v = refs[3 * ns:4 * ns]
        slots = refs[4 * ns:]
        w = lax.axis_index("s") * 2 + lax.axis_index("c")
        base = w * cpw
        for t in range(ns):
            pltpu.sync_copy(i_h[t].at[pl.ds(base, cpw)], i_v[t])

        def issue_g(kk, j):
            bufs, gs = slots[sl * j:sl * j + ns], slots[sl * j + ns]
            for t in range(ns):
                pltpu.async_copy(t_h[t].at[i_v[t].at[kk]], bufs[t], gs)

        def wait_g(j):
            bufs, gs = slots[sl * j:sl * j + ns], slots[sl * j + ns]
            for t in range(ns):
                pltpu.make_async_copy(t_h[t].at[i_v[t].at[0]],
                                      bufs[t], gs).wait()

        def issue_w(kk, j):
            bufs, ws = slots[sl * j:sl * j + ns], slots[sl * j + ns + 1]
            row0 = (base + kk) * chunk
            for t in range(ns):
                pltpu.make_async_copy(
                    bufs[t], o_h[t].at[pl.ds(row0, chunk)], ws).start()

        def wait_w(j):
            bufs, ws = slots[sl * j:sl * j + ns], slots[sl * j + ns + 1]
            for t in range(ns):
                pltpu.make_async_copy(
                    bufs[t], o_h[t].at[pl.ds(0, chunk)], ws).wait()

        for j in range(nslots):
            issue_g(j, j)

        nloop = -(-cpw // nslots)

        @pl.loop(0, nloop)
        def _(i):
            for j in range(nslots):
                kk = i * nslots + j

                @pl.when(kk < cpw)
                def _():
                    wait_g(j)
                    issue_w(kk, j)

                    @pl.when(kk + nslots < cpw)
                    def _():
                        wait_w(j)
                        issue_g(kk + nslots, j)

        for j in range(nslots):
            wait_w(j)

    return k(*tabs, *idxs)


def _sc_scatter(e_arr, w_arr, idx2, zeros):
    mesh = plsc.VectorSubcoreMesh(core_axis_name="c", subcore_axis_name="s")
    out_type = [jax.ShapeDtypeStruct((_N, _C), _f32),
                jax.ShapeDtypeStruct((_N, _C), _f32)]

    nslots = 2
    scr = [pltpu.VMEM_SHARED((_N, _C), _f32),
           pltpu.VMEM((_SEGC, _GCH), jnp.int32)]
    for _ in range(nslots):
        scr += [pltpu.VMEM((_GCH, _C), _f32),
                pltpu.SemaphoreType.DMA,
                pltpu.SemaphoreType.DMA]

    @functools.partial(
        pl.kernel, mesh=mesh, out_type=out_type, scratch_types=scr)
    def k(e_h, w_h, idx_h, z_h, s_out, a_out, spm, idx_v, *slots):
        c = lax.axis_index("c")
        sid = lax.axis_index("s")
        pltpu.sync_copy(z_h.at[pl.ds(sid * _ZR, _ZR)],
                        spm.at[pl.ds(sid * _ZR, _ZR)])

        @pl.when(sid == 15)
        def _():
            pltpu.sync_copy(z_h.at[pl.ds(16 * _ZR, _N - 16 * _ZR)],
                            spm.at[pl.ds(16 * _ZR, _N - 16 * _ZR)])

        plsc.subcore_barrier()

        def scat(src_h):
            def issue_l(kk, j):
                b, ls, _ = slots[3 * j:3 * j + 3]
                row0 = (sid * _SPS + kk) * _GCH
                pltpu.async_copy(src_h.at[pl.ds(row0, _GCH)], b, ls)

            def wait_l(j):
                b, ls, _ = slots[3 * j:3 * j + 3]
                pltpu.make_async_copy(src_h.at[pl.ds(0, _GCH)], b, ls).wait()

            def issue_s(q, j):
                b, _, ss = slots[3 * j:3 * j + 3]
                pltpu.async_copy(b, spm.at[idx_v.at[q]], ss, add=True)

            def wait_s(j):
                b, _, ss = slots[3 * j:3 * j + 3]
                pltpu.make_async_copy(b, spm.at[idx_v.at[0]], ss).wait()

            for seg in range(_NSEG):
                pltpu.sync_copy(
                    idx_h.at[pl.ds(sid * _SPS + seg * _SEGC, _SEGC)], idx_v)
                for j in range(nslots):
                    issue_l(seg * _SEGC + j, j)

                @pl.loop(0, _SEGC // nslots)
                def _(i):
                    for j in range(nslots):
                        q = i * nslots + j
                        kk = seg * _SEGC + q
                        wait_l(j)
                        issue_s(q, j)

                        @pl.when(q + nslots < _SEGC)
                        def _():
                            wait_s(j)
                            issue_l(kk + nslots, j)

                for j in range(nslots):
                    wait_s(j)

        @pl.when(c == 0)
        def _():
            scat(e_h)

        @pl.when(c == 1)
        def _():
            scat(w_h)

        plsc.subcore_barrier()

        def writeback(out_h):
            pltpu.sync_copy(spm.at[pl.ds(sid * _ZR, _ZR)],
                            out_h.at[pl.ds(sid * _ZR, _ZR)])

            @pl.when(sid == 15)
            def _():
                pltpu.sync_copy(spm.at[pl.ds(16 * _ZR, _N - 16 * _ZR)],
                                out_h.at[pl.ds(16 * _ZR, _N - 16 * _ZR)])

        @pl.when(c == 0)
        def _():
            writeback(s_out)

        @pl.when(c == 1)
        def _():
            writeback(a_out)

    return k(e_arr, w_arr, idx2, zeros)


def _bn_affine(st, g, be):
    mu = st[0] / _E
    var = st[1] / _E - mu * mu
    scale = g * lax.rsqrt(var + 1e-5)
    shift = be - mu * scale
    return scale, shift


def kernel(x, pos, edge_index, W_lin, W_src, W_dst,
           pW1, pb1, pg1, pbe1, pW2, pb2, pg2, pbe2,
           aW1, ab1, ag1, abe1, aW2, ab2, ag2, abe2,
           up_W, up_b):
    src_p = jnp.pad(edge_index[0], (0, _EP - _E))
    dst_p = jnp.pad(edge_index[1], (0, _EP - _E))
    src128 = src_p.reshape(_GNC, _GCH)
    dst128 = dst_p.reshape(_GNC, _GCH)
    pos16 = jnp.pad(pos, ((0, 0), (0, 13)))
    w1p = jnp.pad(pW1, ((0, 13), (0, 0)))
    wcat = jnp.concatenate([W_dst, W_src, W_lin], axis=1)

    td, ts = _prep(x, pos16, wcat, w1p)

    gd, = _gather([td], [dst128], _GCH, 4)
    gs, = _gather([ts], [src128], _GCH, 3)

    st1 = _pass1(gd, gs, pb1.reshape(1, _C))
    sc1, sh1 = _bn_affine(st1, pg1, pbe1)
    s1 = jnp.stack([sc1, pb1 * sc1 + sh1])

    h2, st2 = _pass2(gd, gs, s1, pW2, pb2.reshape(1, _C))
    sc2, sh2 = _bn_affine(st2, pg2, pbe2)
    s2 = jnp.stack([sc2, sh2])

    h3, st3 = _pass3(h2, gd, gs, s2, aW1, ab1.reshape(1, _C))
    sc3, sh3 = _bn_affine(st3, ag1, abe1)
    s3 = jnp.stack([sc3, sh3])

    h4, st4 = _pass4(h3, s3, aW2, ab2.reshape(1, _C))
    sc4, sh4 = _bn_affine(st4, ag2, abe2)
    s4 = jnp.stack([sc4, sh4])

    e, w = _pass5(h4, h2, gs, s4, s2)

    zeros = jnp.zeros((_N, _C), _f32)
    s_sum, acc = _sc_scatter(e, w, dst128, zeros)

    return _final(s_sum, acc, x, up_W, up_b.reshape(1, _C))


# speedup vs baseline: 3.6577x; 1.0962x over previous
import functools

import jax
import jax.numpy as jnp
from jax import lax
from jax.experimental import pallas as pl
from jax.experimental.pallas import tpu as pltpu
from jax.experimental.pallas import tpu_sc as plsc

_N = 10000
_E = 320000
_C = 128

_EP = 327680
_B = 8192
_GRID = _EP // _B

_GCH = 128
_GNC = _EP // _GCH
_CPW = _GNC // 32
_GC2 = 64

_SPS = _GNC // 16
_SEGC = 32
_NSEG = _SPS // _SEGC
_ZR = 624

_f32 = jnp.float32
_bf16 = jnp.bfloat16
_u32 = jnp.uint32


def _bf16_bits(a):
    r = jax.lax.bitcast_convert_type(a, _u32)
    return (r + jnp.uint32(0x7FFF) + ((r >> 16) & jnp.uint32(1))) >> 16


def _pack2(a, b):
    return _bf16_bits(a) | (_bf16_bits(b) << 16)


def _lo_f32(w):
    return jax.lax.bitcast_convert_type(w << 16, _f32)


def _hi_f32(w):
    return jax.lax.bitcast_convert_type(w & jnp.uint32(0xFFFF0000), _f32)


def _prep_body(x_ref, pos_ref, w_ref, w1_ref, td_ref, ts_ref):
    xw = jnp.dot(x_ref[...], w_ref[...], preferred_element_type=_f32)
    p = jnp.dot(pos_ref[...], w1_ref[...], preferred_element_type=_f32)
    td_ref[...] = _pack2(xw[:, :_C], p)
    ts_ref[...] = jnp.concatenate(
        [_pack2(xw[:, _C:2 * _C], p),
         jax.lax.bitcast_convert_type(xw[:, 2 * _C:], _u32)], 1)


def _prep(x, pos16, wcat, w1p):
    return pl.pallas_call(
        _prep_body,
        grid=(5,),
        in_specs=[pl.BlockSpec((2000, _C), lambda i: (i, 0)),
                  pl.BlockSpec((2000, 16), lambda i: (i, 0)),
                  pl.BlockSpec((_C, 3 * _C), lambda i: (0, 0)),
                  pl.BlockSpec((16, _C), lambda i: (0, 0))],
        out_specs=[pl.BlockSpec((2000, _C), lambda i: (i, 0)),
                   pl.BlockSpec((2000, 2 * _C), lambda i: (i, 0))],
        out_shape=[jax.ShapeDtypeStruct((_N, _C), _u32),
                   jax.ShapeDtypeStruct((_N, 2 * _C), _u32)],
    )(x, pos16, wcat, w1p)


def _edge_mask():
    rows = lax.broadcasted_iota(jnp.int32, (_B, 1), 0) + pl.program_id(0) * _B
    return (rows < _E).astype(_f32)


def _acc_stats(st_ref, h):
    m = _edge_mask()
    hm = h * m
    blk = jnp.concatenate([jnp.sum(hm, 0, keepdims=True),
                           jnp.sum(hm * h, 0, keepdims=True)])

    @pl.when(pl.program_id(0) == 0)
    def _():
        st_ref[...] = jnp.zeros_like(st_ref)

    st_ref[...] += blk


def _p1_body(gpd_ref, gps_ref, b_ref, st_ref):
    h = _hi_f32(gpd_ref[...]) - _hi_f32(gps_ref[...]) + b_ref[...]
    _acc_stats(st_ref, h)


def _pass1(gpd, gps, b1):
    return pl.pallas_call(
        _p1_body,
        grid=(_GRID,),
        in_specs=[pl.BlockSpec((_B, _C), lambda i: (i, 0)),
                  pl.BlockSpec((_B, _C), lambda i: (i, 0)),
                  pl.BlockSpec((1, _C), lambda i: (0, 0))],
        out_specs=pl.BlockSpec((2, _C), lambda i: (0, 0)),
        out_shape=jax.ShapeDtypeStruct((2, _C), _f32),
    )(gpd, gps, b1)


def _p2_body(gpd_ref, gps_ref, s1_ref, w2_ref, b2_ref,
             h2_ref, st_ref):
    r = jnp.maximum((_hi_f32(gpd_ref[...]) - _hi_f32(gps_ref[...]))
                    * s1_ref[0:1] + s1_ref[1:2], 0.0)
    h2 = jnp.dot(r, w2_ref[...], preferred_element_type=_f32) + b2_ref[...]
    h2_ref[...] = h2.astype(_bf16)
    _acc_stats(st_ref, h2)


def _pass2(gpd, gps, s1, w2, b2):
    return pl.pallas_call(
        _p2_body,
        grid=(_GRID,),
        in_specs=[pl.BlockSpec((_B, _C), lambda i: (i, 0)),
                  pl.BlockSpec((_B, _C), lambda i: (i, 0)),
                  pl.BlockSpec((2, _C), lambda i: (0, 0)),
                  pl.BlockSpec((_C, _C), lambda i: (0, 0)),
                  pl.BlockSpec((1, _C), lambda i: (0, 0))],
        out_specs=[pl.BlockSpec((_B, _C), lambda i: (i, 0)),
                   pl.BlockSpec((2, _C), lambda i: (0, 0))],
        out_shape=[jax.ShapeDtypeStruct((_EP, _C), _bf16),
                   jax.ShapeDtypeStruct((2, _C), _f32)],
    )(gpd, gps, s1, w2, b2)


def _p3_body(h2_ref, gad_ref, gas_ref, s2_ref, w_ref, b_ref,
             h3_ref, st_ref):
    delta = jnp.maximum(
        h2_ref[...].astype(_f32) * s2_ref[0:1] + s2_ref[1:2], 0.0)
    a0 = _lo_f32(gad_ref[...]) - _lo_f32(gas_ref[...]) + delta
    h3 = jnp.dot(a0, w_ref[...], preferred_element_type=_f32) + b_ref[...]
    h3_ref[...] = h3.astype(_bf16)
    _acc_stats(st_ref, h3)


def _pass3(h2, gad, gas, s2, w, b):
    return pl.pallas_call(
        _p3_body,
        grid=(_GRID,),
        in_specs=[pl.BlockSpec((_B, _C), lambda i: (i, 0)),
                  pl.BlockSpec((_B, _C), lambda i: (i, 0)),
                  pl.BlockSpec((_B, _C), lambda i: (i, 0)),
                  pl.BlockSpec((2, _C), lambda i: (0, 0)),
                  pl.BlockSpec((_C, _C), lambda i: (0, 0)),
                  pl.BlockSpec((1, _C), lambda i: (0, 0))],
        out_specs=[pl.BlockSpec((_B, _C), lambda i: (i, 0)),
                   pl.BlockSpec((2, _C), lambda i: (0, 0))],
        out_shape=[jax.ShapeDtypeStruct((_EP, _C), _bf16),
                   jax.ShapeDtypeStruct((2, _C), _f32)],
    )(h2, gad, gas, s2, w, b)


def _p4_body(h3_ref, s3_ref, w_ref, b_ref, h4_ref, st_ref):
    s = jnp.maximum(
        h3_ref[...].astype(_f32) * s3_ref[0:1] + s3_ref[1:2], 0.0)
    h4 = jnp.dot(s, w_ref[...], preferred_element_type=_f32) + b_ref[...]
    h4_ref[...] = h4.astype(_bf16)
    _acc_stats(st_ref, h4)


def _pass4(h3, s3, w, b):
    return pl.pallas_call(
        _p4_body,
        grid=(_GRID,),
        in_specs=[pl.BlockSpec((_B, _C), lambda i: (i, 0)),
                  pl.BlockSpec((2, _C), lambda i: (0, 0)),
                  pl.BlockSpec((_C, _C), lambda i: (0, 0)),
                  pl.BlockSpec((1, _C), lambda i: (0, 0))],
        out_specs=[pl.BlockSpec((_B, _C), lambda i: (i, 0)),
                   pl.BlockSpec((2, _C), lambda i: (0, 0))],
        out_shape=[jax.ShapeDtypeStruct((_EP, _C), _bf16),
                   jax.ShapeDtypeStruct((2, _C), _f32)],
    )(h3, s3, w, b)


def _p5_body(h4_ref, h2_ref, gv_ref, s4_ref, s2_ref, e_ref, w_ref):
    m = _edge_mask()
    alpha = jnp.maximum(
        h4_ref[...].astype(_f32) * s4_ref[0:1] + s4_ref[1:2], 0.0)
    e = jnp.exp(alpha) * m
    delta = jnp.maximum(
        h2_ref[...].astype(_f32) * s2_ref[0:1] + s2_ref[1:2], 0.0)
    e_ref[...] = e
    w_ref[...] = e * (jax.lax.bitcast_convert_type(gv_ref[...], _f32)
                      + delta)


def _pass5(h4, h2, gv, s4, s2):
    return pl.pallas_call(
        _p5_body,
        grid=(_GRID,),
        in_specs=[pl.BlockSpec((_B, _C), lambda i: (i, 0)),
                  pl.BlockSpec((_B, _C), lambda i: (i, 0)),
                  pl.BlockSpec((_B, _C), lambda i: (i, 1)),
                  pl.BlockSpec((2, _C), lambda i: (0, 0)),
                  pl.BlockSpec((2, _C), lambda i: (0, 0))],
        out_specs=[pl.BlockSpec((_B, _C), lambda i: (i, 0)),
                   pl.BlockSpec((_B, _C), lambda i: (i, 0))],
        out_shape=[jax.ShapeDtypeStruct((_EP, _C), _f32),
                   jax.ShapeDtypeStruct((_EP, _C), _f32)],
    )(h4, h2, gv, s4, s2)


def _fin_body(s_ref, acc_ref, x_ref, w_ref, b_ref, o_ref):
    o = acc_ref[...] / (s_ref[...] + 1e-16)
    o_ref[...] = jnp.dot(o, w_ref[...],
                         preferred_element_type=_f32) + b_ref[...] + x_ref[...]


def _final(s, acc, x, w, b):
    return pl.pallas_call(
        _fin_body,
        grid=(5,),
        in_specs=[pl.BlockSpec((2000, _C), lambda i: (i, 0)),
                  pl.BlockSpec((2000, _C), lambda i: (i, 0)),
                  pl.BlockSpec((2000, _C), lambda i: (i, 0)),
                  pl.BlockSpec((_C, _C), lambda i: (0, 0)),
                  pl.BlockSpec((1, _C), lambda i: (0, 0))],
        out_specs=pl.BlockSpec((2000, _C), lambda i: (i, 0)),
        out_shape=jax.ShapeDtypeStruct((_N, _C), _f32),
    )(s, acc, x, w, b)


def _gather(tabs, idxs, chunk, nslots):
    ns = len(tabs)
    rows = [t.shape[1:] for t in tabs]
    dts = [t.dtype for t in tabs]
    cpw = (_EP // chunk) // 32
    mesh = plsc.VectorSubcoreMesh(core_axis_name="c", subcore_axis_name="s")
    out_type = [jax.ShapeDtypeStruct((_EP,) + r, d) for r, d in zip(rows, dts)]
    scr = [pltpu.VMEM((cpw, chunk), jnp.int32) for _ in range(ns)]
    for _ in range(nslots):
        scr += [pltpu.VMEM((chunk,) + r, d) for r, d in zip(rows, dts)]
        scr += [pltpu.SemaphoreType.DMA, pltpu.SemaphoreType.DMA]

    sl = ns + 2

    @functools.partial(pl.kernel, mesh=mesh, out_type=out_type,
                       scratch_types=scr)
    def k(*refs):
        t_h = refs[:ns]
        i_h = refs[ns:2 * ns]
        o_h = refs[2 * ns:3 * ns]
        i_v = refs[3 * ns:4 * ns]
        slots = refs[4 * ns:]
        w = lax.axis_index("s") * 2 + lax.axis_index("c")
        base = w * cpw
        for t in range(ns):
            pltpu.sync_copy(i_h[t].at[pl.ds(base, cpw)], i_v[t])

        def issue_g(kk, j):
            bufs, gs = slots[sl * j:sl * j + ns], slots[sl * j + ns]
            for t in range(ns):
                pltpu.async_copy(t_h[t].at[i_v[t].at[kk]], bufs[t], gs)

        def wait_g(j):
            bufs, gs = slots[sl * j:sl * j + ns], slots[sl * j + ns]
            for t in range(ns):
                pltpu.make_async_copy(t_h[t].at[i_v[t].at[0]],
                                      bufs[t], gs).wait()

        def issue_w(kk, j):
            bufs, ws = slots[sl * j:sl * j + ns], slots[sl * j + ns + 1]
            row0 = (base + kk) * chunk
            for t in range(ns):
                pltpu.make_async_copy(
                    bufs[t], o_h[t].at[pl.ds(row0, chunk)], ws).start()

        def wait_w(j):
            bufs, ws = slots[sl * j:sl * j + ns], slots[sl * j + ns + 1]
            for t in range(ns):
                pltpu.make_async_copy(
                    bufs[t], o_h[t].at[pl.ds(0, chunk)], ws).wait()

        for j in range(nslots):
            issue_g(j, j)

        nloop = -(-cpw // nslots)

        @pl.loop(0, nloop)
        def _(i):
            for j in range(nslots):
                kk = i * nslots + j

                @pl.when(kk < cpw)
                def _():
                    wait_g(j)
                    issue_w(kk, j)

                    @pl.when(kk + nslots < cpw)
                    def _():
                        wait_w(j)
                        issue_g(kk + nslots, j)

        for j in range(nslots):
            wait_w(j)

    return k(*tabs, *idxs)


def _sc_scatter(e_arr, w_arr, idx2, zeros):
    mesh = plsc.VectorSubcoreMesh(core_axis_name="c", subcore_axis_name="s")
    out_type = [jax.ShapeDtypeStruct((_N, _C), _f32),
                jax.ShapeDtypeStruct((_N, _C), _f32)]

    nslots = 2
    scr = [pltpu.VMEM_SHARED((_N, _C), _f32),
           pltpu.VMEM((_SEGC, _GCH), jnp.int32)]
    for _ in range(nslots):
        scr += [pltpu.VMEM((_GCH, _C), _f32),
                pltpu.SemaphoreType.DMA,
                pltpu.SemaphoreType.DMA]

    @functools.partial(
        pl.kernel, mesh=mesh, out_type=out_type, scratch_types=scr)
    def k(e_h, w_h, idx_h, z_h, s_out, a_out, spm, idx_v, *slots):
        c = lax.axis_index("c")
        sid = lax.axis_index("s")
        pltpu.sync_copy(z_h.at[pl.ds(sid * _ZR, _ZR)],
                        spm.at[pl.ds(sid * _ZR, _ZR)])

        @pl.when(sid == 15)
        def _():
            pltpu.sync_copy(z_h.at[pl.ds(16 * _ZR, _N - 16 * _ZR)],
                            spm.at[pl.ds(16 * _ZR, _N - 16 * _ZR)])

        plsc.subcore_barrier()

        def scat(src_h):
            def issue_l(kk, j):
                b, ls, _ = slots[3 * j:3 * j + 3]
                row0 = (sid * _SPS + kk) * _GCH
                pltpu.async_copy(src_h.at[pl.ds(row0, _GCH)], b, ls)

            def wait_l(j):
                b, ls, _ = slots[3 * j:3 * j + 3]
                pltpu.make_async_copy(src_h.at[pl.ds(0, _GCH)], b, ls).wait()

            def issue_s(q, j):
                b, _, ss = slots[3 * j:3 * j + 3]
                pltpu.async_copy(b, spm.at[idx_v.at[q]], ss, add=True)

            def wait_s(j):
                b, _, ss = slots[3 * j:3 * j + 3]
                pltpu.make_async_copy(b, spm.at[idx_v.at[0]], ss).wait()

            for seg in range(_NSEG):
                pltpu.sync_copy(
                    idx_h.at[pl.ds(sid * _SPS + seg * _SEGC, _SEGC)], idx_v)
                for j in range(nslots):
                    issue_l(seg * _SEGC + j, j)

                @pl.loop(0, _SEGC // nslots)
                def _(i):
                    for j in range(nslots):
                        q = i * nslots + j
                        kk = seg * _SEGC + q
                        wait_l(j)
                        issue_s(q, j)

                        @pl.when(q + nslots < _SEGC)
                        def _():
                            wait_s(j)
                            issue_l(kk + nslots, j)

                for j in range(nslots):
                    wait_s(j)

        @pl.when(c == 0)
        def _():
            scat(e_h)

        @pl.when(c == 1)
        def _():
            scat(w_h)

        plsc.subcore_barrier()

        def writeback(out_h):
            pltpu.sync_copy(spm.at[pl.ds(sid * _ZR, _ZR)],
                            out_h.at[pl.ds(sid * _ZR, _ZR)])

            @pl.when(sid == 15)
            def _():
                pltpu.sync_copy(spm.at[pl.ds(16 * _ZR, _N - 16 * _ZR)],
                                out_h.at[pl.ds(16 * _ZR, _N - 16 * _ZR)])

        @pl.when(c == 0)
        def _():
            writeback(s_out)

        @pl.when(c == 1)
        def _():
            writeback(a_out)

    return k(e_arr, w_arr, idx2, zeros)


def _bn_affine(st, g, be):
    mu = st[0] / _E
    var = st[1] / _E - mu * mu
    scale = g * lax.rsqrt(var + 1e-5)
    shift = be - mu * scale
    return scale, shift


def kernel(x, pos, edge_index, W_lin, W_src, W_dst,
           pW1, pb1, pg1, pbe1, pW2, pb2, pg2, pbe2,
           aW1, ab1, ag1, abe1, aW2, ab2, ag2, abe2,
           up_W, up_b):
    src_p = jnp.pad(edge_index[0], (0, _EP - _E))
    dst_p = jnp.pad(edge_index[1], (0, _EP - _E))
    src128 = src_p.reshape(_GNC, _GCH)
    dst128 = dst_p.reshape(_GNC, _GCH)
    pos16 = jnp.pad(pos, ((0, 0), (0, 13)))
    w1p = jnp.pad(pW1, ((0, 13), (0, 0)))
    wcat = jnp.concatenate([W_dst, W_src, W_lin], axis=1)

    td, ts = _prep(x, pos16, wcat, w1p)

    gd, = _gather([td], [dst128], _GCH, 4)
    gs, = _gather([ts], [src128], _GCH, 3)

    st1 = _pass1(gd, gs, pb1.reshape(1, _C))
    sc1, sh1 = _bn_affine(st1, pg1, pbe1)
    s1 = jnp.stack([sc1, pb1 * sc1 + sh1])

    h2, st2 = _pass2(gd, gs, s1, pW2, pb2.reshape(1, _C))
    sc2, sh2 = _bn_affine(st2, pg2, pbe2)
    s2 = jnp.stack([sc2, sh2])

    h3, st3 = _pass3(h2, gd, gs, s2, aW1, ab1.reshape(1, _C))
    sc3, sh3 = _bn_affine(st3, ag1, abe1)
    s3 = jnp.stack([sc3, sh3])

    h4, st4 = _pass4(h3, s3, aW2, ab2.reshape(1, _C))
    sc4, sh4 = _bn_affine(st4, ag2, abe2)
    s4 = jnp.stack([sc4, sh4])

    e, w = _pass5(h4, h2, gs, s4, s2)

    zeros = jnp.zeros((_N, _C), _f32)
    s_sum, acc = _sc_scatter(e, w, dst128, zeros)

    return _final(s_sum, acc, x, up_W, up_b.reshape(1, _C))


# speedup vs baseline: 3.6607x; 1.0008x over previous
import functools

import jax
import jax.numpy as jnp
from jax import lax
from jax.experimental import pallas as pl
from jax.experimental.pallas import tpu as pltpu
from jax.experimental.pallas import tpu_sc as plsc

_N = 10000
_E = 320000
_C = 128

_EP = 327680
_B = 8192
_GRID = _EP // _B

_GCH = 128
_GNC = _EP // _GCH
_CPW = _GNC // 32
_GC2 = 64

_SPS = _GNC // 16
_SEGC = 32
_NSEG = _SPS // _SEGC
_ZR = 624

_f32 = jnp.float32
_bf16 = jnp.bfloat16
_u32 = jnp.uint32


def _bf16_bits(a):
    r = jax.lax.bitcast_convert_type(a, _u32)
    return (r + jnp.uint32(0x7FFF) + ((r >> 16) & jnp.uint32(1))) >> 16


def _pack2(a, b):
    return _bf16_bits(a) | (_bf16_bits(b) << 16)


def _lo_f32(w):
    return jax.lax.bitcast_convert_type(w << 16, _f32)


def _hi_f32(w):
    return jax.lax.bitcast_convert_type(w & jnp.uint32(0xFFFF0000), _f32)


def _prep_body(x_ref, pos_ref, w_ref, w1_ref, td_ref, ts_ref):
    xw = jnp.dot(x_ref[...], w_ref[...], preferred_element_type=_f32)
    p = jnp.dot(pos_ref[...], w1_ref[...], preferred_element_type=_f32)
    td_ref[...] = _pack2(xw[:, :_C], p)
    ts_ref[...] = jnp.concatenate(
        [_pack2(xw[:, _C:2 * _C], p),
         jax.lax.bitcast_convert_type(xw[:, 2 * _C:], _u32)], 1)


def _prep(x, pos16, wcat, w1p):
    return pl.pallas_call(
        _prep_body,
        grid=(5,),
        in_specs=[pl.BlockSpec((2000, _C), lambda i: (i, 0)),
                  pl.BlockSpec((2000, 16), lambda i: (i, 0)),
                  pl.BlockSpec((_C, 3 * _C), lambda i: (0, 0)),
                  pl.BlockSpec((16, _C), lambda i: (0, 0))],
        out_specs=[pl.BlockSpec((2000, _C), lambda i: (i, 0)),
                   pl.BlockSpec((2000, 2 * _C), lambda i: (i, 0))],
        out_shape=[jax.ShapeDtypeStruct((_N, _C), _u32),
                   jax.ShapeDtypeStruct((_N, 2 * _C), _u32)],
    )(x, pos16, wcat, w1p)


def _edge_mask():
    rows = lax.broadcasted_iota(jnp.int32, (_B, 1), 0) + pl.program_id(0) * _B
    return (rows < _E).astype(_f32)


def _acc_stats(st_ref, h):
    m = _edge_mask()
    hm = h * m
    blk = jnp.concatenate([jnp.sum(hm, 0, keepdims=True),
                           jnp.sum(hm * h, 0, keepdims=True)])

    @pl.when(pl.program_id(0) == 0)
    def _():
        st_ref[...] = jnp.zeros_like(st_ref)

    st_ref[...] += blk


def _p1_body(gpd_ref, gps_ref, b_ref, st_ref):
    h = _hi_f32(gpd_ref[...]) - _hi_f32(gps_ref[...]) + b_ref[...]
    _acc_stats(st_ref, h)


def _pass1(gpd, gps, b1):
    return pl.pallas_call(
        _p1_body,
        grid=(_GRID,),
        in_specs=[pl.BlockSpec((_B, _C), lambda i: (i, 0)),
                  pl.BlockSpec((_B, _C), lambda i: (i, 0)),
                  pl.BlockSpec((1, _C), lambda i: (0, 0))],
        out_specs=pl.BlockSpec((2, _C), lambda i: (0, 0)),
        out_shape=jax.ShapeDtypeStruct((2, _C), _f32),
    )(gpd, gps, b1)


def _p2_body(gpd_ref, gps_ref, s1_ref, w2_ref, b2_ref,
             h2_ref, st_ref):
    r = jnp.maximum((_hi_f32(gpd_ref[...]) - _hi_f32(gps_ref[...]))
                    * s1_ref[0:1] + s1_ref[1:2], 0.0)
    h2 = jnp.dot(r, w2_ref[...], preferred_element_type=_f32) + b2_ref[...]
    h2_ref[...] = h2.astype(_bf16)
    _acc_stats(st_ref, h2)


def _pass2(gpd, gps, s1, w2, b2):
    return pl.pallas_call(
        _p2_body,
        grid=(_GRID,),
        in_specs=[pl.BlockSpec((_B, _C), lambda i: (i, 0)),
                  pl.BlockSpec((_B, _C), lambda i: (i, 0)),
                  pl.BlockSpec((2, _C), lambda i: (0, 0)),
                  pl.BlockSpec((_C, _C), lambda i: (0, 0)),
                  pl.BlockSpec((1, _C), lambda i: (0, 0))],
        out_specs=[pl.BlockSpec((_B, _C), lambda i: (i, 0)),
                   pl.BlockSpec((2, _C), lambda i: (0, 0))],
        out_shape=[jax.ShapeDtypeStruct((_EP, _C), _bf16),
                   jax.ShapeDtypeStruct((2, _C), _f32)],
    )(gpd, gps, s1, w2, b2)


def _p3_body(h2_ref, gad_ref, gas_ref, s2_ref, w_ref, b_ref,
             h3_ref, st_ref):
    delta = jnp.maximum(
        h2_ref[...].astype(_f32) * s2_ref[0:1] + s2_ref[1:2], 0.0)
    a0 = _lo_f32(gad_ref[...]) - _lo_f32(gas_ref[...]) + delta
    h3 = jnp.dot(a0, w_ref[...], preferred_element_type=_f32) + b_ref[...]
    h3_ref[...] = h3.astype(_bf16)
    _acc_stats(st_ref, h3)


def _pass3(h2, gad, gas, s2, w, b):
    return pl.pallas_call(
        _p3_body,
        grid=(_GRID,),
        in_specs=[pl.BlockSpec((_B, _C), lambda i: (i, 0)),
                  pl.BlockSpec((_B, _C), lambda i: (i, 0)),
                  pl.BlockSpec((_B, _C), lambda i: (i, 0)),
                  pl.BlockSpec((2, _C), lambda i: (0, 0)),
                  pl.BlockSpec((_C, _C), lambda i: (0, 0)),
                  pl.BlockSpec((1, _C), lambda i: (0, 0))],
        out_specs=[pl.BlockSpec((_B, _C), lambda i: (i, 0)),
                   pl.BlockSpec((2, _C), lambda i: (0, 0))],
        out_shape=[jax.ShapeDtypeStruct((_EP, _C), _bf16),
                   jax.ShapeDtypeStruct((2, _C), _f32)],
    )(h2, gad, gas, s2, w, b)


def _p4_body(h3_ref, s3_ref, w_ref, b_ref, h4_ref, st_ref):
    s = jnp.maximum(
        h3_ref[...].astype(_f32) * s3_ref[0:1] + s3_ref[1:2], 0.0)
    h4 = jnp.dot(s, w_ref[...], preferred_element_type=_f32) + b_ref[...]
    h4_ref[...] = h4.astype(_bf16)
    _acc_stats(st_ref, h4)


def _pass4(h3, s3, w, b):
    return pl.pallas_call(
        _p4_body,
        grid=(_GRID,),
        in_specs=[pl.BlockSpec((_B, _C), lambda i: (i, 0)),
                  pl.BlockSpec((2, _C), lambda i: (0, 0)),
                  pl.BlockSpec((_C, _C), lambda i: (0, 0)),
                  pl.BlockSpec((1, _C), lambda i: (0, 0))],
        out_specs=[pl.BlockSpec((_B, _C), lambda i: (i, 0)),
                   pl.BlockSpec((2, _C), lambda i: (0, 0))],
        out_shape=[jax.ShapeDtypeStruct((_EP, _C), _bf16),
                   jax.ShapeDtypeStruct((2, _C), _f32)],
    )(h3, s3, w, b)


def _p5_body(h4_ref, h2_ref, gv_ref, s4_ref, s2_ref, e_ref, w_ref):
    m = _edge_mask()
    alpha = jnp.maximum(
        h4_ref[...].astype(_f32) * s4_ref[0:1] + s4_ref[1:2], 0.0)
    e = jnp.exp(alpha) * m
    delta = jnp.maximum(
        h2_ref[...].astype(_f32) * s2_ref[0:1] + s2_ref[1:2], 0.0)
    e_ref[...] = e
    w_ref[...] = e * (jax.lax.bitcast_convert_type(gv_ref[...], _f32)
                      + delta)


def _pass5(h4, h2, gv, s4, s2):
    return pl.pallas_call(
        _p5_body,
        grid=(_GRID,),
        in_specs=[pl.BlockSpec((_B, _C), lambda i: (i, 0)),
                  pl.BlockSpec((_B, _C), lambda i: (i, 0)),
                  pl.BlockSpec((_B, _C), lambda i: (i, 1)),
                  pl.BlockSpec((2, _C), lambda i: (0, 0)),
                  pl.BlockSpec((2, _C), lambda i: (0, 0))],
        out_specs=[pl.BlockSpec((_B, _C), lambda i: (i, 0)),
                   pl.BlockSpec((_B, _C), lambda i: (i, 0))],
        out_shape=[jax.ShapeDtypeStruct((_EP, _C), _f32),
                   jax.ShapeDtypeStruct((_EP, _C), _f32)],
    )(h4, h2, gv, s4, s2)


def _fin_body(s_ref, acc_ref, x_ref, w_ref, b_ref, o_ref):
    o = acc_ref[...] / (s_ref[...] + 1e-16)
    o_ref[...] = jnp.dot(o, w_ref[...],
                         preferred_element_type=_f32) + b_ref[...] + x_ref[...]


def _final(s, acc, x, w, b):
    return pl.pallas_call(
        _fin_body,
        grid=(5,),
        in_specs=[pl.BlockSpec((2000, _C), lambda i: (i, 0)),
                  pl.BlockSpec((2000, _C), lambda i: (i, 0)),
                  pl.BlockSpec((2000, _C), lambda i: (i, 0)),
                  pl.BlockSpec((_C, _C), lambda i: (0, 0)),
                  pl.BlockSpec((1, _C), lambda i: (0, 0))],
        out_specs=pl.BlockSpec((2000, _C), lambda i: (i, 0)),
        out_shape=jax.ShapeDtypeStruct((_N, _C), _f32),
    )(s, acc, x, w, b)


def _gather(tabs, idxs, chunk, nslots):
    ns = len(tabs)
    rows = [t.shape[1:] for t in tabs]
    dts = [t.dtype for t in tabs]
    cpw = (_EP // chunk) // 32
    mesh = plsc.VectorSubcoreMesh(core_axis_name="c", subcore_axis_name="s")
    out_type = [jax.ShapeDtypeStruct((_EP,) + r, d) for r, d in zip(rows, dts)]
    scr = [pltpu.VMEM((cpw, chunk), jnp.int32) for _ in range(ns)]
    for _ in range(nslots):
        scr += [pltpu.VMEM((chunk,) + r, d) for r, d in zip(rows, dts)]
        scr += [pltpu.SemaphoreType.DMA, pltpu.SemaphoreType.DMA]

    sl = ns + 2

    @functools.partial(pl.kernel, mesh=mesh, out_type=out_type,
                       scratch_types=scr)
    def k(*refs):
        t_h = refs[:ns]
        i_h = refs[ns:2 * ns]
        o_h = refs[2 * ns:3 * ns]
        i_v = refs[3 * ns:4 * ns]
        slots = refs[4 * ns:]
        w = lax.axis_index("s") * 2 + lax.axis_index("c")
        base = w * cpw
        for t in range(ns):
            pltpu.sync_copy(i_h[t].at[pl.ds(base, cpw)], i_v[t])

        def issue_g(kk, j):
            bufs, gs = slots[sl * j:sl * j + ns], slots[sl * j + ns]
            for t in range(ns):
                pltpu.async_copy(t_h[t].at[i_v[t].at[kk]], bufs[t], gs)

        def wait_g(j):
            bufs, gs = slots[sl * j:sl * j + ns], slots[sl * j + ns]
            for t in range(ns):
                pltpu.make_async_copy(t_h[t].at[i_v[t].at[0]],
                                      bufs[t], gs).wait()

        def issue_w(kk, j):
            bufs, ws = slots[sl * j:sl * j + ns], slots[sl * j + ns + 1]
            row0 = (base + kk) * chunk
            for t in range(ns):
                pltpu.make_async_copy(
                    bufs[t], o_h[t].at[pl.ds(row0, chunk)], ws).start()

        def wait_w(j):
            bufs, ws = slots[sl * j:sl * j + ns], slots[sl * j + ns + 1]
            for t in range(ns):
                pltpu.make_async_copy(
                    bufs[t], o_h[t].at[pl.ds(0, chunk)], ws).wait()

        for j in range(nslots):
            issue_g(j, j)

        nloop = -(-cpw // nslots)

        @pl.loop(0, nloop)
        def _(i):
            for j in range(nslots):
                kk = i * nslots + j

                @pl.when(kk < cpw)
                def _():
                    wait_g(j)
                    issue_w(kk, j)

                    @pl.when(kk + nslots < cpw)
                    def _():
                        wait_w(j)
                        issue_g(kk + nslots, j)

        for j in range(nslots):
            wait_w(j)

    return k(*tabs, *idxs)


def _sc_scatter(e_arr, w_arr, idx2, zeros):
    mesh = plsc.VectorSubcoreMesh(core_axis_name="c", subcore_axis_name="s")
    out_type = [jax.ShapeDtypeStruct((_N, _C), _f32),
                jax.ShapeDtypeStruct((_N, _C), _f32)]

    nslots = 2
    scr = [pltpu.VMEM_SHARED((_N, _C), _f32),
           pltpu.VMEM((_SEGC, _GCH), jnp.int32)]
    for _ in range(nslots):
        scr += [pltpu.VMEM((_GCH, _C), _f32),
                pltpu.SemaphoreType.DMA,
                pltpu.SemaphoreType.DMA]

    @functools.partial(
        pl.kernel, mesh=mesh, out_type=out_type, scratch_types=scr)
    def k(e_h, w_h, idx_h, z_h, s_out, a_out, spm, idx_v, *slots):
        c = lax.axis_index("c")
        sid = lax.axis_index("s")
        pltpu.sync_copy(z_h.at[pl.ds(sid * _ZR, _ZR)],
                        spm.at[pl.ds(sid * _ZR, _ZR)])

        @pl.when(sid == 15)
        def _():
            pltpu.sync_copy(z_h.at[pl.ds(16 * _ZR, _N - 16 * _ZR)],
                            spm.at[pl.ds(16 * _ZR, _N - 16 * _ZR)])

        plsc.subcore_barrier()

        def scat(src_h):
            def issue_l(kk, j):
                b, ls, _ = slots[3 * j:3 * j + 3]
                row0 = (sid * _SPS + kk) * _GCH
                pltpu.async_copy(src_h.at[pl.ds(row0, _GCH)], b, ls)

            def wait_l(j):
                b, ls, _ = slots[3 * j:3 * j + 3]
                pltpu.make_async_copy(src_h.at[pl.ds(0, _GCH)], b, ls).wait()

            def issue_s(q, j):
                b, _, ss = slots[3 * j:3 * j + 3]
                pltpu.async_copy(b, spm.at[idx_v.at[q]], ss, add=True)

            def wait_s(j):
                b, _, ss = slots[3 * j:3 * j + 3]
                pltpu.make_async_copy(b, spm.at[idx_v.at[0]], ss).wait()

            for seg in range(_NSEG):
                pltpu.sync_copy(
                    idx_h.at[pl.ds(sid * _SPS + seg * _SEGC, _SEGC)], idx_v)
                for j in range(nslots):
                    issue_l(seg * _SEGC + j, j)

                @pl.loop(0, _SEGC // nslots)
                def _(i):
                    for j in range(nslots):
                        q = i * nslots + j
                        kk = seg * _SEGC + q
                        wait_l(j)
                        issue_s(q, j)

                        @pl.when(q + nslots < _SEGC)
                        def _():
                            wait_s(j)
                            issue_l(kk + nslots, j)

                for j in range(nslots):
                    wait_s(j)

        @pl.when(c == 0)
        def _():
            scat(e_h)

        @pl.when(c == 1)
        def _():
            scat(w_h)

        plsc.subcore_barrier()

        def writeback(out_h):
            pltpu.sync_copy(spm.at[pl.ds(sid * _ZR, _ZR)],
                            out_h.at[pl.ds(sid * _ZR, _ZR)])

            @pl.when(sid == 15)
            def _():
                pltpu.sync_copy(spm.at[pl.ds(16 * _ZR, _N - 16 * _ZR)],
                                out_h.at[pl.ds(16 * _ZR, _N - 16 * _ZR)])

        @pl.when(c == 0)
        def _():
            writeback(s_out)

        @pl.when(c == 1)
        def _():
            writeback(a_out)

    return k(e_arr, w_arr, idx2, zeros)


def _bn_affine(st, g, be):
    mu = st[0] / _E
    var = st[1] / _E - mu * mu
    scale = g * lax.rsqrt(var + 1e-5)
    shift = be - mu * scale
    return scale, shift


def kernel(x, pos, edge_index, W_lin, W_src, W_dst,
           pW1, pb1, pg1, pbe1, pW2, pb2, pg2, pbe2,
           aW1, ab1, ag1, abe1, aW2, ab2, ag2, abe2,
           up_W, up_b):
    src_p = jnp.pad(edge_index[0], (0, _EP - _E))
    dst_p = jnp.pad(edge_index[1], (0, _EP - _E))
    src128 = src_p.reshape(_GNC, _GCH)
    dst128 = dst_p.reshape(_GNC, _GCH)
    pos16 = jnp.pad(pos, ((0, 0), (0, 13)))
    w1p = jnp.pad(pW1, ((0, 13), (0, 0)))
    wcat = jnp.concatenate([W_dst, W_src, W_lin], axis=1)

    td, ts = _prep(x, pos16, wcat, w1p)

    gd, = _gather([td], [dst128], _GCH, 5)
    gs, = _gather([ts], [src128], _GCH, 3)

    st1 = _pass1(gd, gs, pb1.reshape(1, _C))
    sc1, sh1 = _bn_affine(st1, pg1, pbe1)
    s1 = jnp.stack([sc1, pb1 * sc1 + sh1])

    h2, st2 = _pass2(gd, gs, s1, pW2, pb2.reshape(1, _C))
    sc2, sh2 = _bn_affine(st2, pg2, pbe2)
    s2 = jnp.stack([sc2, sh2])

    h3, st3 = _pass3(h2, gd, gs, s2, aW1, ab1.reshape(1, _C))
    sc3, sh3 = _bn_affine(st3, ag1, abe1)
    s3 = jnp.stack([sc3, sh3])

    h4, st4 = _pass4(h3, s3, aW2, ab2.reshape(1, _C))
    sc4, sh4 = _bn_affine(st4, ag2, abe2)
    s4 = jnp.stack([sc4, sh4])

    e, w = _pass5(h4, h2, gs, s4, s2)

    zeros = jnp.zeros((_N, _C), _f32)
    s_sum, acc = _sc_scatter(e, w, dst128, zeros)

    return _final(s_sum, acc, x, up_W, up_b.reshape(1, _C))
